# Initial kernel scaffold; baseline (speedup 1.0000x reference)
import jax
import jax.numpy as jnp
from jax.experimental import pallas as pl


def kernel(y_source_oh, source_oh, flow, neg_flow):
    raise NotImplementedError("write your pallas kernel here")


# speedup vs baseline: 4.6049x; 4.6049x over previous
import functools

import jax
import jax.numpy as jnp
from jax import lax
from jax.experimental import pallas as pl
from jax.experimental.pallas import tpu as pltpu
from jax.experimental.pallas import tpu_sc as plsc

H, W, D = 64, 128, 128
NROW = H * W
NBLK = 64
RPB = NROW // NBLK
NCH = 3
NS = 16
NVOX = H * W * D


def _stats_body(y_ref, s_ref, rowcnt_ref, mom_ref, blk_ref):
    b = pl.program_id(0)

    @pl.when(b == 0)
    def _():
        for ch in range(NCH):
            for q in range(8):
                mom_ref[ch, q] = 0.0

    jj = lax.broadcasted_iota(jnp.int32, (RPB, D), 0).astype(jnp.float32)
    kk = lax.broadcasted_iota(jnp.int32, (RPB, D), 1).astype(jnp.float32)
    fi = b.astype(jnp.float32)
    for ch in range(NCH):
        yb = y_ref[ch + 1]
        sb = s_ref[ch + 1]
        cy = jnp.sum(yb)
        cs = jnp.sum(sb)
        rowcnt_ref[0, ch, :] = jnp.sum(yb, axis=1)
        blk_ref[ch, b] = cy
        mom_ref[ch, 0] += cy
        mom_ref[ch, 1] += fi * cy
        mom_ref[ch, 2] += jnp.sum(jj * yb)
        mom_ref[ch, 3] += jnp.sum(kk * yb)
        mom_ref[ch, 4] += cs
        mom_ref[ch, 5] += fi * cs
        mom_ref[ch, 6] += jnp.sum(jj * sb)
        mom_ref[ch, 7] += jnp.sum(kk * sb)


def _run_stats(y4, s4):
    return pl.pallas_call(
        _stats_body,
        grid=(NBLK,),
        in_specs=[
            pl.BlockSpec((4, RPB, D), lambda b: (0, b, 0)),
            pl.BlockSpec((4, RPB, D), lambda b: (0, b, 0)),
        ],
        out_specs=[
            pl.BlockSpec((1, NCH, D), lambda b: (b, 0, 0)),
            pl.BlockSpec(memory_space=pltpu.SMEM),
            pl.BlockSpec(memory_space=pltpu.SMEM),
        ],
        out_shape=[
            jax.ShapeDtypeStruct((NBLK, NCH, D), jnp.float32),
            jax.ShapeDtypeStruct((NCH, 8), jnp.float32),
            jax.ShapeDtypeStruct((NCH, NBLK), jnp.float32),
        ],
        compiler_params=pltpu.CompilerParams(
            dimension_semantics=("arbitrary",)),
    )(y4, s4)


def _sample_body(rowcnt_hbm, blkcnt_hbm, tgt_hbm, y_hbm, f_hbm,
                 rows_out, cols_out, fv_out,
                 rc_v, bc_v, tg_v, idx_v, yrows_v, frow_v, st_v, sti_v, sem):
    cid = lax.axis_index("c")
    sid = lax.axis_index("s")
    wid = sid * 2 + cid

    @pl.when(wid < NCH)
    def _():
        ch = wid
        pltpu.sync_copy(rowcnt_hbm.at[pl.ds(ch * NROW, NROW)], rc_v)
        pltpu.sync_copy(blkcnt_hbm.at[pl.ds(ch * NBLK, NBLK)], bc_v)
        pltpu.sync_copy(tgt_hbm.at[pl.ds(ch * NS, NS)], tg_v)
        lanes = lax.iota(jnp.int32, 16)
        t = tg_v[...]

        z16 = jnp.zeros((16,), jnp.int32)

        def blk_body(b, carry):
            cum, blk, cb = carry
            v = plsc.load_gather(bc_v, [jnp.broadcast_to(b, (16,))])
            newcum = cum + v
            hit = (newcum >= t) & (cum < t)
            blk = jnp.where(hit, b, blk)
            cb = jnp.where(hit, cum, cb)
            return newcum, blk, cb

        _, blk, cb = lax.fori_loop(0, NBLK, blk_body, (z16, z16, z16))

        rowbase = blk * RPB

        def row_body(r, carry):
            cum, row, rstart = carry
            v = plsc.load_gather(rc_v, [rowbase + r])
            newcum = cum + v
            hit = (newcum >= t) & (cum < t)
            row = jnp.where(hit, rowbase + r, row)
            rstart = jnp.where(hit, cum, rstart)
            return newcum, row, rstart

        _, row, rstart = lax.fori_loop(0, RPB, row_body, (cb, z16, z16))
        t_local = t - rstart

        idx_v[...] = row + (ch + 1) * NROW
        pltpu.async_copy(y_hbm.at[idx_v], yrows_v, sem).wait()

        def col_body(c, carry):
            cum2, col = carry
            cs = jnp.broadcast_to(c, (16,))
            vals = plsc.load_gather(yrows_v, [lanes, cs])
            isnz = vals > 0.5
            cnew = cum2 + isnz.astype(jnp.int32)
            col = jnp.where(isnz & (cnew == t_local), cs, col)
            return cnew, col

        _, col = lax.fori_loop(0, D, col_body, (z16, z16))

        sti_v[...] = row
        pltpu.sync_copy(sti_v, rows_out.at[pl.ds(ch * NS, NS)])
        sti_v[...] = col
        pltpu.sync_copy(sti_v, cols_out.at[pl.ds(ch * NS, NS)])

        for comp in range(3):
            idx_v[...] = row + comp * NROW
            pltpu.async_copy(f_hbm.at[idx_v], frow_v, sem).wait()
            st_v[...] = plsc.load_gather(frow_v, [lanes, col])
            pltpu.sync_copy(st_v, fv_out.at[pl.ds((ch * 3 + comp) * NS, NS)])


def _run_sample(rowcnt_i, blkcnt_i, targets, y2d, f2d):
    mesh = plsc.VectorSubcoreMesh(core_axis_name="c", subcore_axis_name="s")
    fn = pl.kernel(
        _sample_body,
        out_type=[
            jax.ShapeDtypeStruct((NCH * NS,), jnp.int32),
            jax.ShapeDtypeStruct((NCH * NS,), jnp.int32),
            jax.ShapeDtypeStruct((NCH * 3 * NS,), jnp.float32),
        ],
        mesh=mesh,
        compiler_params=pltpu.CompilerParams(needs_layout_passes=False),
        scratch_types=[
            pltpu.VMEM((NROW,), jnp.int32),
            pltpu.VMEM((NBLK,), jnp.int32),
            pltpu.VMEM((NS,), jnp.int32),
            pltpu.VMEM((NS,), jnp.int32),
            pltpu.VMEM((NS, D), jnp.float32),
            pltpu.VMEM((NS, D), jnp.float32),
            pltpu.VMEM((NS,), jnp.float32),
            pltpu.VMEM((NS,), jnp.int32),
            pltpu.SemaphoreType.DMA,
        ],
    )
    return fn(rowcnt_i, blkcnt_i, targets, y2d, f2d)


def _loss_body(y_ref, f_ref, aw_ref, out_ref):
    b = pl.program_id(0)

    @pl.when(b == 0)
    def _():
        out_ref[0, 0] = 0.0

    jj = lax.broadcasted_iota(jnp.int32, (RPB, D), 0).astype(jnp.float32)
    kk = lax.broadcasted_iota(jnp.int32, (RPB, D), 1).astype(jnp.float32)
    fi = b.astype(jnp.float32)
    msk = jnp.zeros((RPB, D), jnp.float32)
    v = [jnp.zeros((RPB, D), jnp.float32) for _ in range(3)]
    for ch in range(NCH):
        w = y_ref[ch + 1] * aw_ref[ch, 12]
        msk = msk + w
        for p in range(3):
            a0 = aw_ref[ch, 4 * p + 0]
            a1 = aw_ref[ch, 4 * p + 1]
            a2 = aw_ref[ch, 4 * p + 2]
            a3 = aw_ref[ch, 4 * p + 3]
            m = (a0 * fi + a3) + a1 * jj + a2 * kk
            v[p] = v[p] + w * m
    s2 = jnp.zeros((RPB, D), jnp.float32)
    for p in range(3):
        r = v[p] - msk * f_ref[p]
        s2 = s2 + r * r
    out_ref[0, 0] += jnp.sum(jnp.sqrt(s2))


def _run_loss(y4, f3, aw):
    return pl.pallas_call(
        _loss_body,
        grid=(NBLK,),
        in_specs=[
            pl.BlockSpec((4, RPB, D), lambda b: (0, b, 0)),
            pl.BlockSpec((3, RPB, D), lambda b: (0, b, 0)),
            pl.BlockSpec(memory_space=pltpu.SMEM),
        ],
        out_specs=pl.BlockSpec(memory_space=pltpu.SMEM),
        out_shape=jax.ShapeDtypeStruct((1, 1), jnp.float32),
        compiler_params=pltpu.CompilerParams(
            dimension_semantics=("arbitrary",)),
    )(y4, f3, aw)


def _rigid_fits(mom, rows, cols, fv):
    cnt_y = mom[:, 0]
    cnt_s = mom[:, 4]
    valid = (cnt_y > 100.0) & (cnt_s > 100.0)
    ty = jnp.where(cnt_y > 0, cnt_y, 1.0)
    ts = jnp.where(cnt_s > 0, cnt_s, 1.0)
    y_cm = jnp.stack([mom[:, 1], mom[:, 2], mom[:, 3]], 1) / ty[:, None]
    s_cm = jnp.stack([mom[:, 5], mom[:, 6], mom[:, 7]], 1) / ts[:, None]

    src = jnp.stack([(rows // W).astype(jnp.float32),
                     (rows % W).astype(jnp.float32),
                     cols.astype(jnp.float32)], 1)
    des = src + fv
    X = src - y_cm[:, :, None]
    Y = des - s_cm[:, :, None]
    Smat = jnp.einsum('cis,cjs->cij', X, Y)
    U, _, Vt = jnp.linalg.svd(Smat)
    V = jnp.transpose(Vt, (0, 2, 1))
    Ut = jnp.transpose(U, (0, 2, 1))
    d = jnp.linalg.det(jnp.matmul(V, Ut))
    diag = jnp.stack([jnp.ones_like(d), jnp.ones_like(d), d], 1)
    R = jnp.matmul(V * diag[:, None, :], Ut)
    t = s_cm - jnp.einsum('cij,cj->ci', R, y_cm)
    A = jnp.concatenate([R - jnp.eye(3, dtype=jnp.float32)[None],
                         t[:, :, None]], axis=2)
    vf = valid.astype(jnp.float32)
    A = A * vf[:, None, None]
    aw = jnp.concatenate(
        [A.reshape(NCH, 12), vf[:, None],
         jnp.zeros((NCH, 3), jnp.float32)], axis=1)
    return aw, valid, cnt_y


def _sample_targets(valid, cnt_y):
    count = cnt_y.astype(jnp.int32)
    rank = jnp.cumsum(valid.astype(jnp.int32)) - 1
    key = jax.random.key(42)
    tgt = []
    for ch in range(NCH):
        kch = jax.random.fold_in(key, rank[ch])
        idx = jax.random.randint(kch, (NS,), 0,
                                 jnp.maximum(count[ch], 1))
        tgt.append(idx + 1)
    return jnp.stack(tgt, 0).astype(jnp.int32)


def kernel(y_source_oh, source_oh, flow, neg_flow):
    y4 = y_source_oh.reshape(4, NROW, D)
    s4 = source_oh.reshape(4, NROW, D)
    f3 = flow.reshape(3, NROW, D)

    rowcnt, mom, blk = _run_stats(y4, s4)

    valid = (mom[:, 0] > 100.0) & (mom[:, 4] > 100.0)
    targets = _sample_targets(valid, mom[:, 0])

    rowcnt_i = jnp.transpose(rowcnt, (1, 0, 2)).reshape(NCH * NROW)
    rowcnt_i = rowcnt_i.astype(jnp.int32)
    blk_i = blk.astype(jnp.int32).reshape(NCH * NBLK)
    y2d = y4.reshape(4 * NROW, D)
    f2d = f3.reshape(3 * NROW, D)
    rows, cols, fv = _run_sample(rowcnt_i, blk_i,
                                 targets.reshape(NCH * NS), y2d, f2d)
    rows = rows.reshape(NCH, NS)
    cols = cols.reshape(NCH, NS)
    fv = fv.reshape(NCH, 3, NS)

    aw, _, _ = _rigid_fits(mom, rows, cols, fv)

    total = _run_loss(y4, f3, aw)
    return (total[0, 0] / NVOX).astype(jnp.float32)


# speedup vs baseline: 6.3615x; 1.3815x over previous
import functools

import jax
import jax.numpy as jnp
from jax import lax
from jax.experimental import pallas as pl
from jax.experimental.pallas import tpu as pltpu
from jax.experimental.pallas import tpu_sc as plsc

H, W, D = 64, 128, 128
NROW = H * W
NBLK = 64
RPB = NROW // NBLK
NGRID = 32
GRB = NROW // NGRID
NCH = 3
NS = 16
NVOX = H * W * D


def _stats_body(y_ref, s_ref, rowcnt_ref, mom_ref, blk_ref):
    b = pl.program_id(0)

    @pl.when(b == 0)
    def _():
        for ch in range(NCH):
            for q in range(8):
                mom_ref[ch, q] = 0.0

    riota = lax.broadcasted_iota(jnp.int32, (GRB, D), 0)
    jj = (riota & (W - 1)).astype(jnp.float32)
    kk = lax.broadcasted_iota(jnp.int32, (GRB, D), 1).astype(jnp.float32)
    bh = b * (GRB // W)
    ii = (lax.shift_right_logical(riota, 7) + bh).astype(jnp.float32)
    for ch in range(NCH):
        yb = y_ref[ch + 1]
        sb = s_ref[ch + 1]
        rs = jnp.sum(yb, axis=1)
        rowcnt_ref[ch, :] = rs
        cy1 = jnp.sum(rs[:RPB])
        cy2 = jnp.sum(rs[RPB:])
        blk_ref[ch, 2 * b] = cy1
        blk_ref[ch, 2 * b + 1] = cy2
        mom_ref[ch, 0] += cy1 + cy2
        mom_ref[ch, 1] += jnp.sum(ii * yb)
        mom_ref[ch, 2] += jnp.sum(jj * yb)
        mom_ref[ch, 3] += jnp.sum(kk * yb)
        mom_ref[ch, 4] += jnp.sum(sb)
        mom_ref[ch, 5] += jnp.sum(ii * sb)
        mom_ref[ch, 6] += jnp.sum(jj * sb)
        mom_ref[ch, 7] += jnp.sum(kk * sb)


def _run_stats(y4, s4):
    return pl.pallas_call(
        _stats_body,
        grid=(NGRID,),
        in_specs=[
            pl.BlockSpec((4, GRB, D), lambda b: (0, b, 0)),
            pl.BlockSpec((4, GRB, D), lambda b: (0, b, 0)),
        ],
        out_specs=[
            pl.BlockSpec((NCH, GRB), lambda b: (0, b)),
            pl.BlockSpec(memory_space=pltpu.SMEM),
            pl.BlockSpec(memory_space=pltpu.SMEM),
        ],
        out_shape=[
            jax.ShapeDtypeStruct((NCH, NROW), jnp.float32),
            jax.ShapeDtypeStruct((NCH, 8), jnp.float32),
            jax.ShapeDtypeStruct((NCH, NBLK), jnp.float32),
        ],
        compiler_params=pltpu.CompilerParams(
            dimension_semantics=("arbitrary",)),
    )(y4, s4)


def _sample_body(rowcnt_hbm, blkcnt_hbm, tgt_hbm, y_hbm, f_hbm,
                 rows_out, cols_out, fv_out,
                 rc_v, bc_v, tg_v, idx_v, yrows_v, frow_v, st_v, sti_v, sem):
    cid = lax.axis_index("c")
    sid = lax.axis_index("s")
    wid = sid * 2 + cid

    @pl.when(wid < NCH)
    def _():
        ch = wid
        pltpu.sync_copy(rowcnt_hbm.at[pl.ds(ch * NROW, NROW)], rc_v)
        pltpu.sync_copy(blkcnt_hbm.at[pl.ds(ch * NBLK, NBLK)], bc_v)
        pltpu.sync_copy(tgt_hbm.at[pl.ds(ch * NS, NS)], tg_v)
        lanes = lax.iota(jnp.int32, 16)
        t = tg_v[...]

        z16 = jnp.zeros((16,), jnp.int32)

        def blk_body(b, carry):
            cum, blk, cb = carry
            v = plsc.load_gather(bc_v, [jnp.broadcast_to(b, (16,))])
            newcum = cum + v
            hit = (newcum >= t) & (cum < t)
            blk = jnp.where(hit, b, blk)
            cb = jnp.where(hit, cum, cb)
            return newcum, blk, cb

        _, blk, cb = lax.fori_loop(0, NBLK, blk_body, (z16, z16, z16))

        rowbase = blk * RPB

        def row_body(r, carry):
            cum, row, rstart = carry
            v = plsc.load_gather(rc_v, [rowbase + r])
            newcum = cum + v
            hit = (newcum >= t) & (cum < t)
            row = jnp.where(hit, rowbase + r, row)
            rstart = jnp.where(hit, cum, rstart)
            return newcum, row, rstart

        _, row, rstart = lax.fori_loop(0, RPB, row_body, (cb, z16, z16))
        t_local = t - rstart

        idx_v[...] = row + (ch + 1) * NROW
        pltpu.async_copy(y_hbm.at[idx_v], yrows_v, sem).wait()

        def col_body(c, carry):
            cum2, col = carry
            cs = jnp.broadcast_to(c, (16,))
            vals = plsc.load_gather(yrows_v, [lanes, cs])
            isnz = vals > 0.5
            cnew = cum2 + isnz.astype(jnp.int32)
            col = jnp.where(isnz & (cnew == t_local), cs, col)
            return cnew, col

        _, col = lax.fori_loop(0, D, col_body, (z16, z16))

        sti_v[...] = row
        pltpu.sync_copy(sti_v, rows_out.at[pl.ds(ch * NS, NS)])
        sti_v[...] = col
        pltpu.sync_copy(sti_v, cols_out.at[pl.ds(ch * NS, NS)])

        for comp in range(3):
            idx_v[...] = row + comp * NROW
            pltpu.async_copy(f_hbm.at[idx_v], frow_v, sem).wait()
            st_v[...] = plsc.load_gather(frow_v, [lanes, col])
            pltpu.sync_copy(st_v, fv_out.at[pl.ds((ch * 3 + comp) * NS, NS)])


def _run_sample(rowcnt_i, blkcnt_i, targets, y2d, f2d):
    mesh = plsc.VectorSubcoreMesh(core_axis_name="c", subcore_axis_name="s")
    fn = pl.kernel(
        _sample_body,
        out_type=[
            jax.ShapeDtypeStruct((NCH * NS,), jnp.int32),
            jax.ShapeDtypeStruct((NCH * NS,), jnp.int32),
            jax.ShapeDtypeStruct((NCH * 3 * NS,), jnp.float32),
        ],
        mesh=mesh,
        compiler_params=pltpu.CompilerParams(needs_layout_passes=False),
        scratch_types=[
            pltpu.VMEM((NROW,), jnp.int32),
            pltpu.VMEM((NBLK,), jnp.int32),
            pltpu.VMEM((NS,), jnp.int32),
            pltpu.VMEM((NS,), jnp.int32),
            pltpu.VMEM((NS, D), jnp.float32),
            pltpu.VMEM((NS, D), jnp.float32),
            pltpu.VMEM((NS,), jnp.float32),
            pltpu.VMEM((NS,), jnp.int32),
            pltpu.SemaphoreType.DMA,
        ],
    )
    return fn(rowcnt_i, blkcnt_i, targets, y2d, f2d)


def _loss_body(y_ref, f_ref, aw_ref, out_ref):
    b = pl.program_id(0)

    @pl.when(b == 0)
    def _():
        out_ref[0, 0] = 0.0

    riota = lax.broadcasted_iota(jnp.int32, (GRB, D), 0)
    jj = (riota & (W - 1)).astype(jnp.float32)
    kk = lax.broadcasted_iota(jnp.int32, (GRB, D), 1).astype(jnp.float32)
    bh = b * (GRB // W)
    ii = (lax.shift_right_logical(riota, 7) + bh).astype(jnp.float32)
    msk = jnp.zeros((GRB, D), jnp.float32)
    v = [jnp.zeros((GRB, D), jnp.float32) for _ in range(3)]
    for ch in range(NCH):
        w = y_ref[ch + 1] * aw_ref[ch, 12]
        msk = msk + w
        for p in range(3):
            a0 = aw_ref[ch, 4 * p + 0]
            a1 = aw_ref[ch, 4 * p + 1]
            a2 = aw_ref[ch, 4 * p + 2]
            a3 = aw_ref[ch, 4 * p + 3]
            m = a0 * ii + a3 + a1 * jj + a2 * kk
            v[p] = v[p] + w * m
    s2 = jnp.zeros((GRB, D), jnp.float32)
    for p in range(3):
        r = v[p] - msk * f_ref[p]
        s2 = s2 + r * r
    out_ref[0, 0] += jnp.sum(jnp.sqrt(s2))


def _run_loss(y4, f3, aw):
    return pl.pallas_call(
        _loss_body,
        grid=(NGRID,),
        in_specs=[
            pl.BlockSpec((4, GRB, D), lambda b: (0, b, 0)),
            pl.BlockSpec((3, GRB, D), lambda b: (0, b, 0)),
            pl.BlockSpec(memory_space=pltpu.SMEM),
        ],
        out_specs=pl.BlockSpec(memory_space=pltpu.SMEM),
        out_shape=jax.ShapeDtypeStruct((1, 1), jnp.float32),
        compiler_params=pltpu.CompilerParams(
            dimension_semantics=("arbitrary",)),
    )(y4, f3, aw)


def _rigid_fits(mom, rows, cols, fv):
    cnt_y = mom[:, 0]
    cnt_s = mom[:, 4]
    valid = (cnt_y > 100.0) & (cnt_s > 100.0)
    ty = jnp.where(cnt_y > 0, cnt_y, 1.0)
    ts = jnp.where(cnt_s > 0, cnt_s, 1.0)
    y_cm = jnp.stack([mom[:, 1], mom[:, 2], mom[:, 3]], 1) / ty[:, None]
    s_cm = jnp.stack([mom[:, 5], mom[:, 6], mom[:, 7]], 1) / ts[:, None]

    src = jnp.stack([(rows // W).astype(jnp.float32),
                     (rows % W).astype(jnp.float32),
                     cols.astype(jnp.float32)], 1)
    des = src + fv
    X = src - y_cm[:, :, None]
    Y = des - s_cm[:, :, None]
    Smat = jnp.einsum('cis,cjs->cij', X, Y)
    U, _, Vt = jnp.linalg.svd(Smat)
    V = jnp.transpose(Vt, (0, 2, 1))
    Ut = jnp.transpose(U, (0, 2, 1))
    d = jnp.linalg.det(jnp.matmul(V, Ut))
    diag = jnp.stack([jnp.ones_like(d), jnp.ones_like(d), d], 1)
    R = jnp.matmul(V * diag[:, None, :], Ut)
    t = s_cm - jnp.einsum('cij,cj->ci', R, y_cm)
    A = jnp.concatenate([R - jnp.eye(3, dtype=jnp.float32)[None],
                         t[:, :, None]], axis=2)
    vf = valid.astype(jnp.float32)
    A = A * vf[:, None, None]
    aw = jnp.concatenate(
        [A.reshape(NCH, 12), vf[:, None],
         jnp.zeros((NCH, 3), jnp.float32)], axis=1)
    return aw, valid, cnt_y


def _sample_targets(valid, cnt_y):
    count = cnt_y.astype(jnp.int32)
    rank = jnp.cumsum(valid.astype(jnp.int32)) - 1
    key = jax.random.key(42)
    keys = jax.vmap(lambda r: jax.random.fold_in(key, r))(rank)
    idx = jax.vmap(
        lambda k, m: jax.random.randint(k, (NS,), 0, m)
    )(keys, jnp.maximum(count, 1))
    return (idx + 1).astype(jnp.int32)


def kernel(y_source_oh, source_oh, flow, neg_flow):
    y4 = y_source_oh.reshape(4, NROW, D)
    s4 = source_oh.reshape(4, NROW, D)
    f3 = flow.reshape(3, NROW, D)

    rowcnt, mom, blk = _run_stats(y4, s4)

    valid = (mom[:, 0] > 100.0) & (mom[:, 4] > 100.0)
    targets = _sample_targets(valid, mom[:, 0])

    rowcnt_i = rowcnt.reshape(NCH * NROW).astype(jnp.int32)
    blk_i = blk.astype(jnp.int32).reshape(NCH * NBLK)
    y2d = y4.reshape(4 * NROW, D)
    f2d = f3.reshape(3 * NROW, D)
    rows, cols, fv = _run_sample(rowcnt_i, blk_i,
                                 targets.reshape(NCH * NS), y2d, f2d)
    rows = rows.reshape(NCH, NS)
    cols = cols.reshape(NCH, NS)
    fv = fv.reshape(NCH, 3, NS)

    aw, _, _ = _rigid_fits(mom, rows, cols, fv)

    total = _run_loss(y4, f3, aw)
    return (total[0, 0] / NVOX).astype(jnp.float32)


# speedup vs baseline: 8.0536x; 1.2660x over previous
import functools

import jax
import jax.numpy as jnp
from jax import lax
from jax.experimental import pallas as pl
from jax.experimental.pallas import tpu as pltpu
from jax.experimental.pallas import tpu_sc as plsc

H, W, D = 64, 128, 128
NROW = H * W
NBLK = 64
RPB = NROW // NBLK
NGRID = 32
GRB = NROW // NGRID
NCH = 3
NS = 16
NVOX = H * W * D


def _stats_body(y_ref, s_ref, rowcnt_ref, mom_ref, blk_ref):
    b = pl.program_id(0)

    @pl.when(b == 0)
    def _():
        for ch in range(NCH):
            for q in range(8):
                mom_ref[ch, q] = 0.0

    riota = lax.broadcasted_iota(jnp.int32, (GRB, D), 0)
    jj = (riota & (W - 1)).astype(jnp.float32)
    kk = lax.broadcasted_iota(jnp.int32, (GRB, D), 1).astype(jnp.float32)
    bh = b * (GRB // W)
    ii = (lax.shift_right_logical(riota, 7) + bh).astype(jnp.float32)
    for ch in range(NCH):
        yb = y_ref[ch + 1]
        sb = s_ref[ch + 1]
        rs = jnp.sum(yb, axis=1)
        rowcnt_ref[ch, :] = rs
        cy1 = jnp.sum(rs[:RPB])
        cy2 = jnp.sum(rs[RPB:])
        blk_ref[ch, 2 * b] = cy1
        blk_ref[ch, 2 * b + 1] = cy2
        mom_ref[ch, 0] += cy1 + cy2
        mom_ref[ch, 1] += jnp.sum(ii * yb)
        mom_ref[ch, 2] += jnp.sum(jj * yb)
        mom_ref[ch, 3] += jnp.sum(kk * yb)
        mom_ref[ch, 4] += jnp.sum(sb)
        mom_ref[ch, 5] += jnp.sum(ii * sb)
        mom_ref[ch, 6] += jnp.sum(jj * sb)
        mom_ref[ch, 7] += jnp.sum(kk * sb)


def _run_stats(y4, s4):
    return pl.pallas_call(
        _stats_body,
        grid=(NGRID,),
        in_specs=[
            pl.BlockSpec((4, GRB, D), lambda b: (0, b, 0)),
            pl.BlockSpec((4, GRB, D), lambda b: (0, b, 0)),
        ],
        out_specs=[
            pl.BlockSpec((NCH, GRB), lambda b: (0, b)),
            pl.BlockSpec(memory_space=pltpu.SMEM),
            pl.BlockSpec(memory_space=pltpu.SMEM),
        ],
        out_shape=[
            jax.ShapeDtypeStruct((NCH, NROW), jnp.float32),
            jax.ShapeDtypeStruct((NCH, 8), jnp.float32),
            jax.ShapeDtypeStruct((NCH, NBLK), jnp.float32),
        ],
        compiler_params=pltpu.CompilerParams(
            dimension_semantics=("arbitrary",)),
    )(y4, s4)


def _sample_body(rowcnt_hbm, blkcnt_hbm, tgt_hbm, y_hbm, f_hbm,
                 rows_out, cols_out, fv_out,
                 rc_v, bc_v, tg_v, idx_v, yrows_v, frow_v, st_v, sti_v, sem):
    cid = lax.axis_index("c")
    sid = lax.axis_index("s")
    wid = sid * 2 + cid

    @pl.when(wid < NCH)
    def _():
        ch = wid
        pltpu.sync_copy(rowcnt_hbm.at[pl.ds(ch * NROW, NROW)], rc_v)
        pltpu.sync_copy(blkcnt_hbm.at[pl.ds(ch * NBLK, NBLK)], bc_v)
        pltpu.sync_copy(tgt_hbm.at[pl.ds(ch * NS, NS)], tg_v)
        lanes = lax.iota(jnp.int32, 16)
        t = tg_v[...]

        z16 = jnp.zeros((16,), jnp.int32)

        def blk_body(b, carry):
            cum, blk, cb = carry
            v = plsc.load_gather(bc_v, [jnp.broadcast_to(b, (16,))])
            newcum = cum + v
            hit = (newcum >= t) & (cum < t)
            blk = jnp.where(hit, b, blk)
            cb = jnp.where(hit, cum, cb)
            return newcum, blk, cb

        _, blk, cb = lax.fori_loop(0, NBLK, blk_body, (z16, z16, z16))

        rowbase = blk * RPB

        def row_body(r, carry):
            cum, row, rstart = carry
            v = plsc.load_gather(rc_v, [rowbase + r])
            newcum = cum + v
            hit = (newcum >= t) & (cum < t)
            row = jnp.where(hit, rowbase + r, row)
            rstart = jnp.where(hit, cum, rstart)
            return newcum, row, rstart

        _, row, rstart = lax.fori_loop(0, RPB, row_body, (cb, z16, z16))
        t_local = t - rstart

        idx_v[...] = row + (ch + 1) * NROW
        pltpu.async_copy(y_hbm.at[idx_v], yrows_v, sem).wait()

        def col_body(c, carry):
            cum2, col = carry
            cs = jnp.broadcast_to(c, (16,))
            vals = plsc.load_gather(yrows_v, [lanes, cs])
            isnz = vals > 0.5
            cnew = cum2 + isnz.astype(jnp.int32)
            col = jnp.where(isnz & (cnew == t_local), cs, col)
            return cnew, col

        _, col = lax.fori_loop(0, D, col_body, (z16, z16))

        sti_v[...] = row
        pltpu.sync_copy(sti_v, rows_out.at[pl.ds(ch * NS, NS)])
        sti_v[...] = col
        pltpu.sync_copy(sti_v, cols_out.at[pl.ds(ch * NS, NS)])

        for comp in range(3):
            idx_v[...] = row + comp * NROW
            pltpu.async_copy(f_hbm.at[idx_v], frow_v, sem).wait()
            st_v[...] = plsc.load_gather(frow_v, [lanes, col])
            pltpu.sync_copy(st_v, fv_out.at[pl.ds((ch * 3 + comp) * NS, NS)])


def _run_sample(rowcnt_i, blkcnt_i, targets, y2d, f2d):
    mesh = plsc.VectorSubcoreMesh(core_axis_name="c", subcore_axis_name="s")
    fn = pl.kernel(
        _sample_body,
        out_type=[
            jax.ShapeDtypeStruct((NCH * NS,), jnp.int32),
            jax.ShapeDtypeStruct((NCH * NS,), jnp.int32),
            jax.ShapeDtypeStruct((NCH * 3 * NS,), jnp.float32),
        ],
        mesh=mesh,
        compiler_params=pltpu.CompilerParams(needs_layout_passes=False),
        scratch_types=[
            pltpu.VMEM((NROW,), jnp.int32),
            pltpu.VMEM((NBLK,), jnp.int32),
            pltpu.VMEM((NS,), jnp.int32),
            pltpu.VMEM((NS,), jnp.int32),
            pltpu.VMEM((NS, D), jnp.float32),
            pltpu.VMEM((NS, D), jnp.float32),
            pltpu.VMEM((NS,), jnp.float32),
            pltpu.VMEM((NS,), jnp.int32),
            pltpu.SemaphoreType.DMA,
        ],
    )
    return fn(rowcnt_i, blkcnt_i, targets, y2d, f2d)


def _kabsch_prologue(smp_ref, cm_ref, aw_ref):
    one = jnp.float32(1.0)
    zero = jnp.float32(0.0)
    for ch in range(NCH):
        ycm = [cm_ref[ch, 0], cm_ref[ch, 1], cm_ref[ch, 2]]
        scm = [cm_ref[ch, 3], cm_ref[ch, 4], cm_ref[ch, 5]]
        vw = cm_ref[ch, 6]
        X = [smp_ref[ch, p, :] - ycm[p] for p in range(3)]
        Y = [smp_ref[ch, p, :] + smp_ref[ch, 3 + p, :] - scm[p]
             for p in range(3)]
        S = [[jnp.sum(X[p] * Y[q]) for q in range(3)] for p in range(3)]
        detS = (S[0][0] * (S[1][1] * S[2][2] - S[1][2] * S[2][1])
                - S[0][1] * (S[1][0] * S[2][2] - S[1][2] * S[2][0])
                + S[0][2] * (S[1][0] * S[2][1] - S[1][1] * S[2][0]))
        d = jnp.where(detS >= 0.0, one, -one)
        K = [[S[0][a] * S[0][b] + S[1][a] * S[1][b] + S[2][a] * S[2][b]
              for b in range(3)] for a in range(3)]
        V = [[one if i == j else zero for j in range(3)] for i in range(3)]
        for _ in range(6):
            for (p, q) in ((0, 1), (0, 2), (1, 2)):
                apq = K[p][q]
                app = K[p][p]
                aqq = K[q][q]
                apq_s = jnp.where(jnp.abs(apq) > 1e-30, apq,
                                  jnp.float32(1e-30))
                tau = (aqq - app) / (2.0 * apq_s)
                tt = jnp.sign(tau) / (jnp.abs(tau)
                                      + jnp.sqrt(1.0 + tau * tau))
                c = 1.0 / jnp.sqrt(1.0 + tt * tt)
                s = tt * c
                r = 3 - p - q
                akp = K[r][p]
                akq = K[r][q]
                K[r][p] = K[p][r] = c * akp - s * akq
                K[r][q] = K[q][r] = s * akp + c * akq
                K[p][p] = app - tt * apq
                K[q][q] = aqq + tt * apq
                K[p][q] = K[q][p] = zero
                for i in range(3):
                    vip = V[i][p]
                    viq = V[i][q]
                    V[i][p] = c * vip - s * viq
                    V[i][q] = s * vip + c * viq
        lam = [jnp.maximum(K[p][p], 0.0) for p in range(3)]
        inv = [jnp.sqrt(lam[p]) / (lam[p] + jnp.float32(1e-30))
               for p in range(3)]
        U = [[(S[i][0] * V[0][p] + S[i][1] * V[1][p] + S[i][2] * V[2][p])
              * inv[p] for p in range(3)] for i in range(3)]
        vm = [jnp.where(lam[0] <= lam[1],
                        jnp.where(lam[0] <= lam[2], V[i][0], V[i][2]),
                        jnp.where(lam[1] <= lam[2], V[i][1], V[i][2]))
              for i in range(3)]
        um = [jnp.where(lam[0] <= lam[1],
                        jnp.where(lam[0] <= lam[2], U[i][0], U[i][2]),
                        jnp.where(lam[1] <= lam[2], U[i][1], U[i][2]))
              for i in range(3)]
        R = [[(V[i][0] * U[j][0] + V[i][1] * U[j][1] + V[i][2] * U[j][2])
              - (one - d) * vm[i] * um[j] for j in range(3)]
             for i in range(3)]
        for p in range(3):
            tp = scm[p] - (R[p][0] * ycm[0] + R[p][1] * ycm[1]
                           + R[p][2] * ycm[2])
            for q in range(3):
                rr = R[p][q] - (one if p == q else zero)
                aw_ref[ch, 4 * p + q] = rr * vw
            aw_ref[ch, 4 * p + 3] = tp * vw
        aw_ref[ch, 12] = vw


def _loss_body(y_ref, f_ref, smp_ref, cm_ref, out_ref, aw_ref):
    b = pl.program_id(0)

    @pl.when(b == 0)
    def _():
        out_ref[0, 0] = 0.0
        _kabsch_prologue(smp_ref, cm_ref, aw_ref)

    riota = lax.broadcasted_iota(jnp.int32, (GRB, D), 0)
    jj = (riota & (W - 1)).astype(jnp.float32)
    kk = lax.broadcasted_iota(jnp.int32, (GRB, D), 1).astype(jnp.float32)
    bh = b * (GRB // W)
    ii = (lax.shift_right_logical(riota, 7) + bh).astype(jnp.float32)
    msk = jnp.zeros((GRB, D), jnp.float32)
    v = [jnp.zeros((GRB, D), jnp.float32) for _ in range(3)]
    for ch in range(NCH):
        w = y_ref[ch + 1] * aw_ref[ch, 12]
        msk = msk + w
        for p in range(3):
            a0 = aw_ref[ch, 4 * p + 0]
            a1 = aw_ref[ch, 4 * p + 1]
            a2 = aw_ref[ch, 4 * p + 2]
            a3 = aw_ref[ch, 4 * p + 3]
            m = a0 * ii + a3 + a1 * jj + a2 * kk
            v[p] = v[p] + w * m
    s2 = jnp.zeros((GRB, D), jnp.float32)
    for p in range(3):
        r = v[p] - msk * f_ref[p]
        s2 = s2 + r * r
    out_ref[0, 0] += jnp.sum(jnp.sqrt(s2))


def _run_loss(y4, f3, smp, cm):
    return pl.pallas_call(
        _loss_body,
        grid=(NGRID,),
        in_specs=[
            pl.BlockSpec((4, GRB, D), lambda b: (0, b, 0)),
            pl.BlockSpec((3, GRB, D), lambda b: (0, b, 0)),
            pl.BlockSpec((NCH, 8, NS), lambda b: (0, 0, 0)),
            pl.BlockSpec(memory_space=pltpu.SMEM),
        ],
        out_specs=pl.BlockSpec(memory_space=pltpu.SMEM),
        out_shape=jax.ShapeDtypeStruct((1, 1), jnp.float32),
        scratch_shapes=[pltpu.SMEM((NCH, 16), jnp.float32)],
        compiler_params=pltpu.CompilerParams(
            dimension_semantics=("arbitrary",)),
    )(y4, f3, smp, cm)


def _pack_fit_inputs(mom, rows, cols, fv):
    cnt_y = mom[:, 0]
    cnt_s = mom[:, 4]
    valid = (cnt_y > 100.0) & (cnt_s > 100.0)
    ty = jnp.where(cnt_y > 0, cnt_y, 1.0)
    ts = jnp.where(cnt_s > 0, cnt_s, 1.0)
    y_cm = jnp.stack([mom[:, 1], mom[:, 2], mom[:, 3]], 1) / ty[:, None]
    s_cm = jnp.stack([mom[:, 5], mom[:, 6], mom[:, 7]], 1) / ts[:, None]

    src = jnp.stack([(rows // W).astype(jnp.float32),
                     (rows % W).astype(jnp.float32),
                     cols.astype(jnp.float32)], 1)
    smp = jnp.concatenate([src, fv, jnp.zeros((NCH, 2, NS), jnp.float32)],
                          axis=1)
    vf = valid.astype(jnp.float32)
    cm = jnp.concatenate([y_cm, s_cm, vf[:, None],
                          jnp.zeros((NCH, 1), jnp.float32)], axis=1)
    return smp, cm


def _sample_targets(valid, cnt_y):
    count = cnt_y.astype(jnp.int32)
    rank = jnp.cumsum(valid.astype(jnp.int32)) - 1
    key = jax.random.key(42)
    keys = jax.vmap(lambda r: jax.random.fold_in(key, r))(rank)
    idx = jax.vmap(
        lambda k, m: jax.random.randint(k, (NS,), 0, m)
    )(keys, jnp.maximum(count, 1))
    return (idx + 1).astype(jnp.int32)


def kernel(y_source_oh, source_oh, flow, neg_flow):
    y4 = y_source_oh.reshape(4, NROW, D)
    s4 = source_oh.reshape(4, NROW, D)
    f3 = flow.reshape(3, NROW, D)

    rowcnt, mom, blk = _run_stats(y4, s4)

    valid = (mom[:, 0] > 100.0) & (mom[:, 4] > 100.0)
    targets = _sample_targets(valid, mom[:, 0])

    rowcnt_i = rowcnt.reshape(NCH * NROW).astype(jnp.int32)
    blk_i = blk.astype(jnp.int32).reshape(NCH * NBLK)
    y2d = y4.reshape(4 * NROW, D)
    f2d = f3.reshape(3 * NROW, D)
    rows, cols, fv = _run_sample(rowcnt_i, blk_i,
                                 targets.reshape(NCH * NS), y2d, f2d)
    rows = rows.reshape(NCH, NS)
    cols = cols.reshape(NCH, NS)
    fv = fv.reshape(NCH, 3, NS)

    smp, cm = _pack_fit_inputs(mom, rows, cols, fv)

    total = _run_loss(y4, f3, smp, cm)
    return (total[0, 0] / NVOX).astype(jnp.float32)


# speedup vs baseline: 8.1782x; 1.0155x over previous
import functools

import jax
import jax.numpy as jnp
from jax import lax
from jax.experimental import pallas as pl
from jax.experimental.pallas import tpu as pltpu
from jax.experimental.pallas import tpu_sc as plsc

H, W, D = 64, 128, 128
NROW = H * W
NBLK = 64
RPB = NROW // NBLK
NGRID = 32
GRB = NROW // NGRID
NCH = 3
NS = 16
NVOX = H * W * D


def _stats_body(y_ref, s_ref, rowcnt_ref, mom_ref, blk_ref):
    b = pl.program_id(0)

    @pl.when(b == 0)
    def _():
        for ch in range(NCH):
            for q in range(8):
                mom_ref[ch, q] = 0.0

    ri = lax.broadcasted_iota(jnp.int32, (GRB,), 0)
    jr = (ri & (W - 1)).astype(jnp.float32)
    ir = (lax.shift_right_logical(ri, 7) + b * (GRB // W)).astype(jnp.float32)
    kv = lax.broadcasted_iota(jnp.int32, (D,), 0).astype(jnp.float32)
    for ch in range(NCH):
        yb = y_ref[ch + 1]
        sb = s_ref[ch + 1]
        rs = jnp.sum(yb, axis=1)
        rss = jnp.sum(sb, axis=1)
        rowcnt_ref[ch, :] = rs.astype(jnp.int32)
        cy1 = jnp.sum(rs[:RPB])
        cy2 = jnp.sum(rs[RPB:])
        blk_ref[ch, 2 * b] = cy1.astype(jnp.int32)
        blk_ref[ch, 2 * b + 1] = cy2.astype(jnp.int32)
        mom_ref[ch, 0] += cy1 + cy2
        mom_ref[ch, 1] += jnp.sum(ir * rs)
        mom_ref[ch, 2] += jnp.sum(jr * rs)
        mom_ref[ch, 3] += jnp.sum(jnp.sum(yb, axis=0) * kv)
        mom_ref[ch, 4] += jnp.sum(rss)
        mom_ref[ch, 5] += jnp.sum(ir * rss)
        mom_ref[ch, 6] += jnp.sum(jr * rss)
        mom_ref[ch, 7] += jnp.sum(jnp.sum(sb, axis=0) * kv)


def _run_stats(y4, s4):
    return pl.pallas_call(
        _stats_body,
        grid=(NGRID,),
        in_specs=[
            pl.BlockSpec((4, GRB, D), lambda b: (0, b, 0)),
            pl.BlockSpec((4, GRB, D), lambda b: (0, b, 0)),
        ],
        out_specs=[
            pl.BlockSpec((NCH, GRB), lambda b: (0, b)),
            pl.BlockSpec(memory_space=pltpu.SMEM),
            pl.BlockSpec(memory_space=pltpu.SMEM),
        ],
        out_shape=[
            jax.ShapeDtypeStruct((NCH, NROW), jnp.int32),
            jax.ShapeDtypeStruct((NCH, 8), jnp.float32),
            jax.ShapeDtypeStruct((NCH, NBLK), jnp.int32),
        ],
        compiler_params=pltpu.CompilerParams(
            dimension_semantics=("arbitrary",)),
    )(y4, s4)


def _sample_body(rowcnt_hbm, blkcnt_hbm, tgt_hbm, y_hbm, f_hbm,
                 rows_out, cols_out, fv_out,
                 rc_v, bc_v, tg_v, idx_v, yrows_v, frow_v, st_v, sti_v, sem):
    cid = lax.axis_index("c")
    sid = lax.axis_index("s")
    wid = sid * 2 + cid

    @pl.when(wid < NCH)
    def _():
        ch = wid
        pltpu.sync_copy(rowcnt_hbm.at[pl.ds(ch * NROW, NROW)], rc_v)
        pltpu.sync_copy(blkcnt_hbm.at[pl.ds(ch * NBLK, NBLK)], bc_v)
        pltpu.sync_copy(tgt_hbm.at[pl.ds(ch * NS, NS)], tg_v)
        lanes = lax.iota(jnp.int32, 16)
        t = tg_v[...]

        z16 = jnp.zeros((16,), jnp.int32)

        def blk_body(b, carry):
            cum, blk, cb = carry
            v = plsc.load_gather(bc_v, [jnp.broadcast_to(b, (16,))])
            newcum = cum + v
            hit = (newcum >= t) & (cum < t)
            blk = jnp.where(hit, b, blk)
            cb = jnp.where(hit, cum, cb)
            return newcum, blk, cb

        _, blk, cb = lax.fori_loop(0, NBLK, blk_body, (z16, z16, z16))

        rowbase = blk * RPB

        def row_body(r, carry):
            cum, row, rstart = carry
            v = plsc.load_gather(rc_v, [rowbase + r])
            newcum = cum + v
            hit = (newcum >= t) & (cum < t)
            row = jnp.where(hit, rowbase + r, row)
            rstart = jnp.where(hit, cum, rstart)
            return newcum, row, rstart

        _, row, rstart = lax.fori_loop(0, RPB, row_body, (cb, z16, z16))
        t_local = t - rstart

        idx_v[...] = row + (ch + 1) * NROW
        pltpu.async_copy(y_hbm.at[idx_v], yrows_v, sem).wait()

        def col_body(c, carry):
            cum2, col = carry
            cs = jnp.broadcast_to(c, (16,))
            vals = plsc.load_gather(yrows_v, [lanes, cs])
            isnz = vals > 0.5
            cnew = cum2 + isnz.astype(jnp.int32)
            col = jnp.where(isnz & (cnew == t_local), cs, col)
            return cnew, col

        _, col = lax.fori_loop(0, D, col_body, (z16, z16))

        sti_v[...] = row
        pltpu.sync_copy(sti_v, rows_out.at[pl.ds(ch * NS, NS)])
        sti_v[...] = col
        pltpu.sync_copy(sti_v, cols_out.at[pl.ds(ch * NS, NS)])

        for comp in range(3):
            idx_v[...] = row + comp * NROW
            pltpu.async_copy(f_hbm.at[idx_v], frow_v, sem).wait()
            st_v[...] = plsc.load_gather(frow_v, [lanes, col])
            pltpu.sync_copy(st_v, fv_out.at[pl.ds((ch * 3 + comp) * NS, NS)])


def _run_sample(rowcnt_i, blkcnt_i, targets, y2d, f2d):
    mesh = plsc.VectorSubcoreMesh(core_axis_name="c", subcore_axis_name="s")
    fn = pl.kernel(
        _sample_body,
        out_type=[
            jax.ShapeDtypeStruct((NCH * NS,), jnp.int32),
            jax.ShapeDtypeStruct((NCH * NS,), jnp.int32),
            jax.ShapeDtypeStruct((NCH * 3 * NS,), jnp.float32),
        ],
        mesh=mesh,
        compiler_params=pltpu.CompilerParams(needs_layout_passes=False),
        scratch_types=[
            pltpu.VMEM((NROW,), jnp.int32),
            pltpu.VMEM((NBLK,), jnp.int32),
            pltpu.VMEM((NS,), jnp.int32),
            pltpu.VMEM((NS,), jnp.int32),
            pltpu.VMEM((NS, D), jnp.float32),
            pltpu.VMEM((NS, D), jnp.float32),
            pltpu.VMEM((NS,), jnp.float32),
            pltpu.VMEM((NS,), jnp.int32),
            pltpu.SemaphoreType.DMA,
        ],
    )
    return fn(rowcnt_i, blkcnt_i, targets, y2d, f2d)


def _kabsch_prologue(mom_ref, rows_ref, cols_ref, fv_ref, aw_ref):
    one = jnp.float32(1.0)
    zero = jnp.float32(0.0)
    for ch in range(NCH):
        cnt_y = mom_ref[ch, 0]
        cnt_s = mom_ref[ch, 4]
        vw = jnp.where((cnt_y > 100.0) & (cnt_s > 100.0), one, zero)
        ty = jnp.where(cnt_y > 0.0, cnt_y, one)
        ts = jnp.where(cnt_s > 0.0, cnt_s, one)
        ycm = [mom_ref[ch, 1 + p] / ty for p in range(3)]
        scm = [mom_ref[ch, 5 + p] / ts for p in range(3)]
        rowv = rows_ref[pl.ds(ch * NS, NS)]
        colv = cols_ref[pl.ds(ch * NS, NS)]
        src = [lax.shift_right_logical(rowv, 7).astype(jnp.float32),
               (rowv & (W - 1)).astype(jnp.float32),
               colv.astype(jnp.float32)]
        X = [src[p] - ycm[p] for p in range(3)]
        Y = [src[p] + fv_ref[pl.ds((ch * 3 + p) * NS, NS)] - scm[p]
             for p in range(3)]
        S = [[jnp.sum(X[p] * Y[q]) for q in range(3)] for p in range(3)]
        detS = (S[0][0] * (S[1][1] * S[2][2] - S[1][2] * S[2][1])
                - S[0][1] * (S[1][0] * S[2][2] - S[1][2] * S[2][0])
                + S[0][2] * (S[1][0] * S[2][1] - S[1][1] * S[2][0]))
        d = jnp.where(detS >= 0.0, one, -one)
        K = [[S[0][a] * S[0][b] + S[1][a] * S[1][b] + S[2][a] * S[2][b]
              for b in range(3)] for a in range(3)]
        V = [[one if i == j else zero for j in range(3)] for i in range(3)]
        for _ in range(6):
            for (p, q) in ((0, 1), (0, 2), (1, 2)):
                apq = K[p][q]
                app = K[p][p]
                aqq = K[q][q]
                apq_s = jnp.where(jnp.abs(apq) > 1e-30, apq,
                                  jnp.float32(1e-30))
                tau = (aqq - app) / (2.0 * apq_s)
                tt = jnp.sign(tau) / (jnp.abs(tau)
                                      + jnp.sqrt(1.0 + tau * tau))
                c = 1.0 / jnp.sqrt(1.0 + tt * tt)
                s = tt * c
                r = 3 - p - q
                akp = K[r][p]
                akq = K[r][q]
                K[r][p] = K[p][r] = c * akp - s * akq
                K[r][q] = K[q][r] = s * akp + c * akq
                K[p][p] = app - tt * apq
                K[q][q] = aqq + tt * apq
                K[p][q] = K[q][p] = zero
                for i in range(3):
                    vip = V[i][p]
                    viq = V[i][q]
                    V[i][p] = c * vip - s * viq
                    V[i][q] = s * vip + c * viq
        lam = [jnp.maximum(K[p][p], 0.0) for p in range(3)]
        inv = [jnp.sqrt(lam[p]) / (lam[p] + jnp.float32(1e-30))
               for p in range(3)]
        U = [[(S[i][0] * V[0][p] + S[i][1] * V[1][p] + S[i][2] * V[2][p])
              * inv[p] for p in range(3)] for i in range(3)]
        vm = [jnp.where(lam[0] <= lam[1],
                        jnp.where(lam[0] <= lam[2], V[i][0], V[i][2]),
                        jnp.where(lam[1] <= lam[2], V[i][1], V[i][2]))
              for i in range(3)]
        um = [jnp.where(lam[0] <= lam[1],
                        jnp.where(lam[0] <= lam[2], U[i][0], U[i][2]),
                        jnp.where(lam[1] <= lam[2], U[i][1], U[i][2]))
              for i in range(3)]
        R = [[(V[i][0] * U[j][0] + V[i][1] * U[j][1] + V[i][2] * U[j][2])
              - (one - d) * vm[i] * um[j] for j in range(3)]
             for i in range(3)]
        for p in range(3):
            tp = scm[p] - (R[p][0] * ycm[0] + R[p][1] * ycm[1]
                           + R[p][2] * ycm[2])
            for q in range(3):
                rr = R[p][q] - (one if p == q else zero)
                aw_ref[ch, 4 * p + q] = rr * vw
            aw_ref[ch, 4 * p + 3] = tp * vw
        aw_ref[ch, 12] = vw


def _loss_body(y_ref, f_ref, mom_ref, rows_ref, cols_ref, fv_ref,
               out_ref, aw_ref):
    b = pl.program_id(0)

    @pl.when(b == 0)
    def _():
        out_ref[0, 0] = 0.0
        _kabsch_prologue(mom_ref, rows_ref, cols_ref, fv_ref, aw_ref)

    ri = lax.broadcasted_iota(jnp.int32, (GRB, 1), 0)
    jcol = (ri & (W - 1)).astype(jnp.float32)
    icol = (lax.shift_right_logical(ri, 7)
            + b * (GRB // W)).astype(jnp.float32)
    krow = lax.broadcasted_iota(jnp.int32, (1, D), 1).astype(jnp.float32)
    msk = jnp.zeros((GRB, D), jnp.float32)
    v = [jnp.zeros((GRB, D), jnp.float32) for _ in range(3)]
    for ch in range(NCH):
        yb = y_ref[ch + 1]
        msk = msk + yb * aw_ref[ch, 12]
        for p in range(3):
            rowpart = (aw_ref[ch, 4 * p + 0] * icol
                       + aw_ref[ch, 4 * p + 1] * jcol
                       + aw_ref[ch, 4 * p + 3])
            m = rowpart + aw_ref[ch, 4 * p + 2] * krow
            v[p] = v[p] + yb * m
    s2 = jnp.zeros((GRB, D), jnp.float32)
    for p in range(3):
        r = v[p] - msk * f_ref[p]
        s2 = s2 + r * r
    out_ref[0, 0] += jnp.sum(jnp.sqrt(s2))


def _run_loss(y4, f3, mom, rows, cols, fv):
    return pl.pallas_call(
        _loss_body,
        grid=(NGRID,),
        in_specs=[
            pl.BlockSpec((4, GRB, D), lambda b: (0, b, 0)),
            pl.BlockSpec((3, GRB, D), lambda b: (0, b, 0)),
            pl.BlockSpec(memory_space=pltpu.SMEM),
            pl.BlockSpec((NCH * NS,), lambda b: (0,)),
            pl.BlockSpec((NCH * NS,), lambda b: (0,)),
            pl.BlockSpec((NCH * 3 * NS,), lambda b: (0,)),
        ],
        out_specs=pl.BlockSpec(memory_space=pltpu.SMEM),
        out_shape=jax.ShapeDtypeStruct((1, 1), jnp.float32),
        scratch_shapes=[pltpu.SMEM((NCH, 16), jnp.float32)],
        compiler_params=pltpu.CompilerParams(
            dimension_semantics=("arbitrary",)),
    )(y4, f3, mom, rows, cols, fv)


def _sample_targets(valid, cnt_y):
    count = cnt_y.astype(jnp.int32)
    rank = jnp.cumsum(valid.astype(jnp.int32)) - 1
    key = jax.random.key(42)
    keys = jax.vmap(lambda r: jax.random.fold_in(key, r))(rank)
    idx = jax.vmap(
        lambda k, m: jax.random.randint(k, (NS,), 0, m)
    )(keys, jnp.maximum(count, 1))
    return (idx + 1).astype(jnp.int32)


def kernel(y_source_oh, source_oh, flow, neg_flow):
    y4 = y_source_oh.reshape(4, NROW, D)
    s4 = source_oh.reshape(4, NROW, D)
    f3 = flow.reshape(3, NROW, D)

    rowcnt, mom, blk = _run_stats(y4, s4)

    valid = (mom[:, 0] > 100.0) & (mom[:, 4] > 100.0)
    targets = _sample_targets(valid, mom[:, 0])

    y2d = y4.reshape(4 * NROW, D)
    f2d = f3.reshape(3 * NROW, D)
    rows, cols, fv = _run_sample(rowcnt.reshape(NCH * NROW),
                                 blk.reshape(NCH * NBLK),
                                 targets.reshape(NCH * NS), y2d, f2d)
    total = _run_loss(y4, f3, mom, rows, cols, fv)
    return (total[0, 0] / NVOX).astype(jnp.float32)


# speedup vs baseline: 8.3975x; 1.0268x over previous
import functools

import jax
import jax.numpy as jnp
from jax import lax
from jax.experimental import pallas as pl
from jax.experimental.pallas import tpu as pltpu
from jax.experimental.pallas import tpu_sc as plsc

H, W, D = 64, 128, 128
NROW = H * W
NBLK = 64
RPB = NROW // NBLK
NGRID = 32
GRB = NROW // NGRID
NCH = 3
NS = 16
NVOX = H * W * D


def _stats_body(y1_ref, y2_ref, y3_ref, s1_ref, s2_ref, s3_ref,
                rowcnt_ref, mom_ref, blk_ref, code_ref):
    b = pl.program_id(0)

    @pl.when(b == 0)
    def _():
        for ch in range(NCH):
            for q in range(8):
                mom_ref[ch, q] = 0.0

    ri = lax.broadcasted_iota(jnp.int32, (GRB,), 0)
    jr = (ri & (W - 1)).astype(jnp.float32)
    ir = (lax.shift_right_logical(ri, 7) + b * (GRB // W)).astype(jnp.float32)
    kv = lax.broadcasted_iota(jnp.int32, (D,), 0).astype(jnp.float32)
    ys = [y1_ref[...], y2_ref[...], y3_ref[...]]
    ss = [s1_ref[...], s2_ref[...], s3_ref[...]]
    code = (ys[0] + 2.0 * ys[1] + 4.0 * ys[2]).astype(jnp.int32)
    code_ref[...] = code.astype(jnp.int8)
    for ch in range(NCH):
        yb = ys[ch]
        sb = ss[ch]
        rs = jnp.sum(yb, axis=1)
        rss = jnp.sum(sb, axis=1)
        rowcnt_ref[ch, :] = rs.astype(jnp.int32)
        cy1 = jnp.sum(rs[:RPB])
        cy2 = jnp.sum(rs[RPB:])
        blk_ref[ch, 2 * b] = cy1.astype(jnp.int32)
        blk_ref[ch, 2 * b + 1] = cy2.astype(jnp.int32)
        mom_ref[ch, 0] += cy1 + cy2
        mom_ref[ch, 1] += jnp.sum(ir * rs)
        mom_ref[ch, 2] += jnp.sum(jr * rs)
        mom_ref[ch, 3] += jnp.sum(jnp.sum(yb, axis=0) * kv)
        mom_ref[ch, 4] += jnp.sum(rss)
        mom_ref[ch, 5] += jnp.sum(ir * rss)
        mom_ref[ch, 6] += jnp.sum(jr * rss)
        mom_ref[ch, 7] += jnp.sum(jnp.sum(sb, axis=0) * kv)


def _chan_spec(ch):
    return pl.BlockSpec((GRB, D), lambda b, c=ch: ((c + 1) * (NROW // GRB) + b, 0))


def _run_stats(y2d, s2d):
    return pl.pallas_call(
        _stats_body,
        grid=(NGRID,),
        in_specs=[_chan_spec(0), _chan_spec(1), _chan_spec(2),
                  _chan_spec(0), _chan_spec(1), _chan_spec(2)],
        out_specs=[
            pl.BlockSpec((NCH, GRB), lambda b: (0, b)),
            pl.BlockSpec(memory_space=pltpu.SMEM),
            pl.BlockSpec(memory_space=pltpu.SMEM),
            pl.BlockSpec((GRB, D), lambda b: (b, 0)),
        ],
        out_shape=[
            jax.ShapeDtypeStruct((NCH, NROW), jnp.int32),
            jax.ShapeDtypeStruct((NCH, 8), jnp.float32),
            jax.ShapeDtypeStruct((NCH, NBLK), jnp.int32),
            jax.ShapeDtypeStruct((NROW, D), jnp.int8),
        ],
        compiler_params=pltpu.CompilerParams(
            dimension_semantics=("arbitrary",)),
    )(y2d, y2d, y2d, s2d, s2d, s2d)


def _sample_body(rowcnt_hbm, blkcnt_hbm, tgt_hbm, y_hbm, f_hbm,
                 rows_out, cols_out, fv_out,
                 rc_v, bc_v, tg_v, idx_v, yrows_v, frow_v, st_v, sti_v, sem):
    cid = lax.axis_index("c")
    sid = lax.axis_index("s")
    wid = sid * 2 + cid

    @pl.when(wid < NCH)
    def _():
        ch = wid
        pltpu.sync_copy(rowcnt_hbm.at[pl.ds(ch * NROW, NROW)], rc_v)
        pltpu.sync_copy(blkcnt_hbm.at[pl.ds(ch * NBLK, NBLK)], bc_v)
        pltpu.sync_copy(tgt_hbm.at[pl.ds(ch * NS, NS)], tg_v)
        lanes = lax.iota(jnp.int32, 16)
        t = tg_v[...]

        z16 = jnp.zeros((16,), jnp.int32)

        def blk_body(b, carry):
            cum, blk, cb = carry
            v = plsc.load_gather(bc_v, [jnp.broadcast_to(b, (16,))])
            newcum = cum + v
            hit = (newcum >= t) & (cum < t)
            blk = jnp.where(hit, b, blk)
            cb = jnp.where(hit, cum, cb)
            return newcum, blk, cb

        _, blk, cb = lax.fori_loop(0, NBLK, blk_body, (z16, z16, z16))

        rowbase = blk * RPB

        def row_body(r, carry):
            cum, row, rstart = carry
            v = plsc.load_gather(rc_v, [rowbase + r])
            newcum = cum + v
            hit = (newcum >= t) & (cum < t)
            row = jnp.where(hit, rowbase + r, row)
            rstart = jnp.where(hit, cum, rstart)
            return newcum, row, rstart

        _, row, rstart = lax.fori_loop(0, RPB, row_body, (cb, z16, z16))
        t_local = t - rstart

        idx_v[...] = row + (ch + 1) * NROW
        pltpu.async_copy(y_hbm.at[idx_v], yrows_v, sem).wait()

        def col_body(c, carry):
            cum2, col = carry
            cs = jnp.broadcast_to(c, (16,))
            vals = plsc.load_gather(yrows_v, [lanes, cs])
            isnz = vals > 0.5
            cnew = cum2 + isnz.astype(jnp.int32)
            col = jnp.where(isnz & (cnew == t_local), cs, col)
            return cnew, col

        _, col = lax.fori_loop(0, D, col_body, (z16, z16))

        sti_v[...] = row
        pltpu.sync_copy(sti_v, rows_out.at[pl.ds(ch * NS, NS)])
        sti_v[...] = col
        pltpu.sync_copy(sti_v, cols_out.at[pl.ds(ch * NS, NS)])

        for comp in range(3):
            idx_v[...] = row + comp * NROW
            pltpu.async_copy(f_hbm.at[idx_v], frow_v, sem).wait()
            st_v[...] = plsc.load_gather(frow_v, [lanes, col])
            pltpu.sync_copy(st_v, fv_out.at[pl.ds((ch * 3 + comp) * NS, NS)])


def _run_sample(rowcnt_i, blkcnt_i, targets, y2d, f2d):
    mesh = plsc.VectorSubcoreMesh(core_axis_name="c", subcore_axis_name="s")
    fn = pl.kernel(
        _sample_body,
        out_type=[
            jax.ShapeDtypeStruct((NCH * NS,), jnp.int32),
            jax.ShapeDtypeStruct((NCH * NS,), jnp.int32),
            jax.ShapeDtypeStruct((NCH * 3 * NS,), jnp.float32),
        ],
        mesh=mesh,
        compiler_params=pltpu.CompilerParams(needs_layout_passes=False),
        scratch_types=[
            pltpu.VMEM((NROW,), jnp.int32),
            pltpu.VMEM((NBLK,), jnp.int32),
            pltpu.VMEM((NS,), jnp.int32),
            pltpu.VMEM((NS,), jnp.int32),
            pltpu.VMEM((NS, D), jnp.float32),
            pltpu.VMEM((NS, D), jnp.float32),
            pltpu.VMEM((NS,), jnp.float32),
            pltpu.VMEM((NS,), jnp.int32),
            pltpu.SemaphoreType.DMA,
        ],
    )
    return fn(rowcnt_i, blkcnt_i, targets, y2d, f2d)


def _kabsch_prologue(mom_ref, rows_ref, cols_ref, fv_ref, aw_ref):
    one = jnp.float32(1.0)
    zero = jnp.float32(0.0)
    for ch in range(NCH):
        cnt_y = mom_ref[ch, 0]
        cnt_s = mom_ref[ch, 4]
        vw = jnp.where((cnt_y > 100.0) & (cnt_s > 100.0), one, zero)
        ty = jnp.where(cnt_y > 0.0, cnt_y, one)
        ts = jnp.where(cnt_s > 0.0, cnt_s, one)
        ycm = [mom_ref[ch, 1 + p] / ty for p in range(3)]
        scm = [mom_ref[ch, 5 + p] / ts for p in range(3)]
        rowv = rows_ref[pl.ds(ch * NS, NS)]
        colv = cols_ref[pl.ds(ch * NS, NS)]
        src = [lax.shift_right_logical(rowv, 7).astype(jnp.float32),
               (rowv & (W - 1)).astype(jnp.float32),
               colv.astype(jnp.float32)]
        X = [src[p] - ycm[p] for p in range(3)]
        Y = [src[p] + fv_ref[pl.ds((ch * 3 + p) * NS, NS)] - scm[p]
             for p in range(3)]
        S = [[jnp.sum(X[p] * Y[q]) for q in range(3)] for p in range(3)]
        detS = (S[0][0] * (S[1][1] * S[2][2] - S[1][2] * S[2][1])
                - S[0][1] * (S[1][0] * S[2][2] - S[1][2] * S[2][0])
                + S[0][2] * (S[1][0] * S[2][1] - S[1][1] * S[2][0]))
        d = jnp.where(detS >= 0.0, one, -one)
        K = [[S[0][a] * S[0][b] + S[1][a] * S[1][b] + S[2][a] * S[2][b]
              for b in range(3)] for a in range(3)]
        V = [[one if i == j else zero for j in range(3)] for i in range(3)]
        for _ in range(6):
            for (p, q) in ((0, 1), (0, 2), (1, 2)):
                apq = K[p][q]
                app = K[p][p]
                aqq = K[q][q]
                apq_s = jnp.where(jnp.abs(apq) > 1e-30, apq,
                                  jnp.float32(1e-30))
                tau = (aqq - app) / (2.0 * apq_s)
                tt = jnp.sign(tau) / (jnp.abs(tau)
                                      + jnp.sqrt(1.0 + tau * tau))
                c = 1.0 / jnp.sqrt(1.0 + tt * tt)
                s = tt * c
                r = 3 - p - q
                akp = K[r][p]
                akq = K[r][q]
                K[r][p] = K[p][r] = c * akp - s * akq
                K[r][q] = K[q][r] = s * akp + c * akq
                K[p][p] = app - tt * apq
                K[q][q] = aqq + tt * apq
                K[p][q] = K[q][p] = zero
                for i in range(3):
                    vip = V[i][p]
                    viq = V[i][q]
                    V[i][p] = c * vip - s * viq
                    V[i][q] = s * vip + c * viq
        lam = [jnp.maximum(K[p][p], 0.0) for p in range(3)]
        inv = [jnp.sqrt(lam[p]) / (lam[p] + jnp.float32(1e-30))
               for p in range(3)]
        U = [[(S[i][0] * V[0][p] + S[i][1] * V[1][p] + S[i][2] * V[2][p])
              * inv[p] for p in range(3)] for i in range(3)]
        vm = [jnp.where(lam[0] <= lam[1],
                        jnp.where(lam[0] <= lam[2], V[i][0], V[i][2]),
                        jnp.where(lam[1] <= lam[2], V[i][1], V[i][2]))
              for i in range(3)]
        um = [jnp.where(lam[0] <= lam[1],
                        jnp.where(lam[0] <= lam[2], U[i][0], U[i][2]),
                        jnp.where(lam[1] <= lam[2], U[i][1], U[i][2]))
              for i in range(3)]
        R = [[(V[i][0] * U[j][0] + V[i][1] * U[j][1] + V[i][2] * U[j][2])
              - (one - d) * vm[i] * um[j] for j in range(3)]
             for i in range(3)]
        for p in range(3):
            tp = scm[p] - (R[p][0] * ycm[0] + R[p][1] * ycm[1]
                           + R[p][2] * ycm[2])
            for q in range(3):
                rr = R[p][q] - (one if p == q else zero)
                aw_ref[ch, 4 * p + q] = rr * vw
            aw_ref[ch, 4 * p + 3] = tp * vw
        aw_ref[ch, 12] = vw


def _loss_body(code_ref, f_ref, mom_ref, rows_ref, cols_ref, fv_ref,
               out_ref, aw_ref):
    b = pl.program_id(0)

    @pl.when(b == 0)
    def _():
        out_ref[0, 0] = 0.0
        _kabsch_prologue(mom_ref, rows_ref, cols_ref, fv_ref, aw_ref)

    ri = lax.broadcasted_iota(jnp.int32, (GRB, 1), 0)
    jcol = (ri & (W - 1)).astype(jnp.float32)
    icol = (lax.shift_right_logical(ri, 7)
            + b * (GRB // W)).astype(jnp.float32)
    krow = lax.broadcasted_iota(jnp.int32, (1, D), 1).astype(jnp.float32)
    code = code_ref[...].astype(jnp.int32)
    msk = jnp.zeros((GRB, D), jnp.float32)
    v = [jnp.zeros((GRB, D), jnp.float32) for _ in range(3)]
    for ch in range(NCH):
        yb = (lax.shift_right_logical(code, ch) & 1).astype(jnp.float32)
        msk = msk + yb * aw_ref[ch, 12]
        for p in range(3):
            rowpart = (aw_ref[ch, 4 * p + 0] * icol
                       + aw_ref[ch, 4 * p + 1] * jcol
                       + aw_ref[ch, 4 * p + 3])
            m = rowpart + aw_ref[ch, 4 * p + 2] * krow
            v[p] = v[p] + yb * m
    s2 = jnp.zeros((GRB, D), jnp.float32)
    for p in range(3):
        r = v[p] - msk * f_ref[p]
        s2 = s2 + r * r
    out_ref[0, 0] += jnp.sum(jnp.sqrt(s2))


def _run_loss(code, f3, mom, rows, cols, fv):
    return pl.pallas_call(
        _loss_body,
        grid=(NGRID,),
        in_specs=[
            pl.BlockSpec((GRB, D), lambda b: (b, 0)),
            pl.BlockSpec((3, GRB, D), lambda b: (0, b, 0)),
            pl.BlockSpec(memory_space=pltpu.SMEM),
            pl.BlockSpec((NCH * NS,), lambda b: (0,)),
            pl.BlockSpec((NCH * NS,), lambda b: (0,)),
            pl.BlockSpec((NCH * 3 * NS,), lambda b: (0,)),
        ],
        out_specs=pl.BlockSpec(memory_space=pltpu.SMEM),
        out_shape=jax.ShapeDtypeStruct((1, 1), jnp.float32),
        scratch_shapes=[pltpu.SMEM((NCH, 16), jnp.float32)],
        compiler_params=pltpu.CompilerParams(
            dimension_semantics=("arbitrary",)),
    )(code, f3, mom, rows, cols, fv)


def _sample_targets(valid, cnt_y):
    count = cnt_y.astype(jnp.int32)
    rank = jnp.cumsum(valid.astype(jnp.int32)) - 1
    key = jax.random.key(42)
    keys = jax.vmap(lambda r: jax.random.fold_in(key, r))(rank)
    idx = jax.vmap(
        lambda k, m: jax.random.randint(k, (NS,), 0, m)
    )(keys, jnp.maximum(count, 1))
    return (idx + 1).astype(jnp.int32)


def kernel(y_source_oh, source_oh, flow, neg_flow):
    y2d = y_source_oh.reshape(4 * NROW, D)
    s2d = source_oh.reshape(4 * NROW, D)
    f3 = flow.reshape(3, NROW, D)
    f2d = flow.reshape(3 * NROW, D)

    rowcnt, mom, blk, code = _run_stats(y2d, s2d)

    valid = (mom[:, 0] > 100.0) & (mom[:, 4] > 100.0)
    targets = _sample_targets(valid, mom[:, 0])

    rows, cols, fv = _run_sample(rowcnt.reshape(NCH * NROW),
                                 blk.reshape(NCH * NBLK),
                                 targets.reshape(NCH * NS), y2d, f2d)
    total = _run_loss(code, f3, mom, rows, cols, fv)
    return (total[0, 0] / NVOX).astype(jnp.float32)


# speedup vs baseline: 9.6992x; 1.1550x over previous
import functools

import jax
import jax.numpy as jnp
from jax import lax
from jax.experimental import pallas as pl
from jax.experimental.pallas import tpu as pltpu
from jax.experimental.pallas import tpu_sc as plsc

H, W, D = 64, 128, 128
NROW = H * W
NBLK = 64
RPB = NROW // NBLK
NGRID = 16
GRB = NROW // NGRID
NCH = 3
NS = 16
NVOX = H * W * D


def _stats_body(y1_ref, y2_ref, y3_ref, s1_ref, s2_ref, s3_ref,
                rowcnt_ref, mom_ref, blk_ref, code_ref):
    b = pl.program_id(0)

    @pl.when(b == 0)
    def _():
        for ch in range(NCH):
            for q in range(8):
                mom_ref[ch, q] = 0.0

    ri = lax.broadcasted_iota(jnp.int32, (GRB,), 0)
    jr = (ri & (W - 1)).astype(jnp.float32)
    ir = (lax.shift_right_logical(ri, 7) + b * (GRB // W)).astype(jnp.float32)
    kv = lax.broadcasted_iota(jnp.int32, (D,), 0).astype(jnp.float32)
    ys = [y1_ref[...], y2_ref[...], y3_ref[...]]
    ss = [s1_ref[...], s2_ref[...], s3_ref[...]]
    code = (ys[0] + 2.0 * ys[1] + 4.0 * ys[2]).astype(jnp.int32)
    code_ref[...] = code.astype(jnp.int8)
    for ch in range(NCH):
        yb = ys[ch]
        sb = ss[ch]
        rs = jnp.sum(yb, axis=1)
        rss = jnp.sum(sb, axis=1)
        rowcnt_ref[ch, :] = rs.astype(jnp.int32)
        cy = jnp.float32(0.0)
        for q in range(GRB // RPB):
            cyp = jnp.sum(rs[q * RPB:(q + 1) * RPB])
            blk_ref[ch, (GRB // RPB) * b + q] = cyp.astype(jnp.int32)
            cy = cy + cyp
        mom_ref[ch, 0] += cy
        mom_ref[ch, 1] += jnp.sum(ir * rs)
        mom_ref[ch, 2] += jnp.sum(jr * rs)
        mom_ref[ch, 3] += jnp.sum(jnp.sum(yb, axis=0) * kv)
        mom_ref[ch, 4] += jnp.sum(rss)
        mom_ref[ch, 5] += jnp.sum(ir * rss)
        mom_ref[ch, 6] += jnp.sum(jr * rss)
        mom_ref[ch, 7] += jnp.sum(jnp.sum(sb, axis=0) * kv)


def _chan_spec(ch):
    return pl.BlockSpec((GRB, D), lambda b, c=ch: ((c + 1) * (NROW // GRB) + b, 0))


def _run_stats(y2d, s2d):
    return pl.pallas_call(
        _stats_body,
        grid=(NGRID,),
        in_specs=[_chan_spec(0), _chan_spec(1), _chan_spec(2),
                  _chan_spec(0), _chan_spec(1), _chan_spec(2)],
        out_specs=[
            pl.BlockSpec((NCH, GRB), lambda b: (0, b)),
            pl.BlockSpec(memory_space=pltpu.SMEM),
            pl.BlockSpec(memory_space=pltpu.SMEM),
            pl.BlockSpec((GRB, D), lambda b: (b, 0)),
        ],
        out_shape=[
            jax.ShapeDtypeStruct((NCH, NROW), jnp.int32),
            jax.ShapeDtypeStruct((NCH, 8), jnp.float32),
            jax.ShapeDtypeStruct((NCH, NBLK), jnp.int32),
            jax.ShapeDtypeStruct((NROW, D), jnp.int8),
        ],
        compiler_params=pltpu.CompilerParams(
            dimension_semantics=("arbitrary",)),
    )(y2d, y2d, y2d, s2d, s2d, s2d)


def _sample_body(rowcnt_hbm, blkcnt_hbm, tgt_hbm, y_hbm, f_hbm,
                 rows_out, cols_out, fv_out,
                 rc_v, bc_v, tg_v, idx_v, yrows_v, frow_v, st_v, sti_v, sem):
    cid = lax.axis_index("c")
    sid = lax.axis_index("s")
    wid = sid * 2 + cid

    @pl.when(wid < NCH)
    def _():
        ch = wid
        pltpu.sync_copy(rowcnt_hbm.at[pl.ds(ch * NROW, NROW)], rc_v)
        pltpu.sync_copy(blkcnt_hbm.at[pl.ds(ch * NBLK, NBLK)], bc_v)
        pltpu.sync_copy(tgt_hbm.at[pl.ds(ch * NS, NS)], tg_v)
        lanes = lax.iota(jnp.int32, 16)
        t = tg_v[...]

        z16 = jnp.zeros((16,), jnp.int32)

        def blk_body(b, carry):
            cum, blk, cb = carry
            v = plsc.load_gather(bc_v, [jnp.broadcast_to(b, (16,))])
            newcum = cum + v
            hit = (newcum >= t) & (cum < t)
            blk = jnp.where(hit, b, blk)
            cb = jnp.where(hit, cum, cb)
            return newcum, blk, cb

        _, blk, cb = lax.fori_loop(0, NBLK, blk_body, (z16, z16, z16))

        rowbase = blk * RPB

        def row_body(r, carry):
            cum, row, rstart = carry
            v = plsc.load_gather(rc_v, [rowbase + r])
            newcum = cum + v
            hit = (newcum >= t) & (cum < t)
            row = jnp.where(hit, rowbase + r, row)
            rstart = jnp.where(hit, cum, rstart)
            return newcum, row, rstart

        _, row, rstart = lax.fori_loop(0, RPB, row_body, (cb, z16, z16))
        t_local = t - rstart

        idx_v[...] = row + (ch + 1) * NROW
        pltpu.async_copy(y_hbm.at[idx_v], yrows_v, sem).wait()

        def col_body(c, carry):
            cum2, col = carry
            cs = jnp.broadcast_to(c, (16,))
            vals = plsc.load_gather(yrows_v, [lanes, cs])
            isnz = vals > 0.5
            cnew = cum2 + isnz.astype(jnp.int32)
            col = jnp.where(isnz & (cnew == t_local), cs, col)
            return cnew, col

        _, col = lax.fori_loop(0, D, col_body, (z16, z16))

        sti_v[...] = row
        pltpu.sync_copy(sti_v, rows_out.at[pl.ds(ch * NS, NS)])
        sti_v[...] = col
        pltpu.sync_copy(sti_v, cols_out.at[pl.ds(ch * NS, NS)])

        for comp in range(3):
            idx_v[...] = row + comp * NROW
            pltpu.async_copy(f_hbm.at[idx_v], frow_v, sem).wait()
            st_v[...] = plsc.load_gather(frow_v, [lanes, col])
            pltpu.sync_copy(st_v, fv_out.at[pl.ds((ch * 3 + comp) * NS, NS)])


def _run_sample(rowcnt_i, blkcnt_i, targets, y2d, f2d):
    mesh = plsc.VectorSubcoreMesh(core_axis_name="c", subcore_axis_name="s")
    fn = pl.kernel(
        _sample_body,
        out_type=[
            jax.ShapeDtypeStruct((NCH * NS,), jnp.int32),
            jax.ShapeDtypeStruct((NCH * NS,), jnp.int32),
            jax.ShapeDtypeStruct((NCH * 3 * NS,), jnp.float32),
        ],
        mesh=mesh,
        compiler_params=pltpu.CompilerParams(needs_layout_passes=False),
        scratch_types=[
            pltpu.VMEM((NROW,), jnp.int32),
            pltpu.VMEM((NBLK,), jnp.int32),
            pltpu.VMEM((NS,), jnp.int32),
            pltpu.VMEM((NS,), jnp.int32),
            pltpu.VMEM((NS, D), jnp.float32),
            pltpu.VMEM((NS, D), jnp.float32),
            pltpu.VMEM((NS,), jnp.float32),
            pltpu.VMEM((NS,), jnp.int32),
            pltpu.SemaphoreType.DMA,
        ],
    )
    return fn(rowcnt_i, blkcnt_i, targets, y2d, f2d)


def _kabsch_prologue(mom_ref, rows_ref, cols_ref, fv_ref, aw_ref):
    one = jnp.float32(1.0)
    zero = jnp.float32(0.0)
    for ch in range(NCH):
        cnt_y = mom_ref[ch, 0]
        cnt_s = mom_ref[ch, 4]
        vw = jnp.where((cnt_y > 100.0) & (cnt_s > 100.0), one, zero)
        ty = jnp.where(cnt_y > 0.0, cnt_y, one)
        ts = jnp.where(cnt_s > 0.0, cnt_s, one)
        ycm = [mom_ref[ch, 1 + p] / ty for p in range(3)]
        scm = [mom_ref[ch, 5 + p] / ts for p in range(3)]
        rowv = rows_ref[pl.ds(ch * NS, NS)]
        colv = cols_ref[pl.ds(ch * NS, NS)]
        src = [lax.shift_right_logical(rowv, 7).astype(jnp.float32),
               (rowv & (W - 1)).astype(jnp.float32),
               colv.astype(jnp.float32)]
        X = [src[p] - ycm[p] for p in range(3)]
        Y = [src[p] + fv_ref[pl.ds((ch * 3 + p) * NS, NS)] - scm[p]
             for p in range(3)]
        S = [[jnp.sum(X[p] * Y[q]) for q in range(3)] for p in range(3)]
        detS = (S[0][0] * (S[1][1] * S[2][2] - S[1][2] * S[2][1])
                - S[0][1] * (S[1][0] * S[2][2] - S[1][2] * S[2][0])
                + S[0][2] * (S[1][0] * S[2][1] - S[1][1] * S[2][0]))
        d = jnp.where(detS >= 0.0, one, -one)
        K = [[S[0][a] * S[0][b] + S[1][a] * S[1][b] + S[2][a] * S[2][b]
              for b in range(3)] for a in range(3)]
        V = [[one if i == j else zero for j in range(3)] for i in range(3)]
        for _ in range(6):
            for (p, q) in ((0, 1), (0, 2), (1, 2)):
                apq = K[p][q]
                app = K[p][p]
                aqq = K[q][q]
                apq_s = jnp.where(jnp.abs(apq) > 1e-30, apq,
                                  jnp.float32(1e-30))
                tau = (aqq - app) / (2.0 * apq_s)
                tt = jnp.sign(tau) / (jnp.abs(tau)
                                      + jnp.sqrt(1.0 + tau * tau))
                c = 1.0 / jnp.sqrt(1.0 + tt * tt)
                s = tt * c
                r = 3 - p - q
                akp = K[r][p]
                akq = K[r][q]
                K[r][p] = K[p][r] = c * akp - s * akq
                K[r][q] = K[q][r] = s * akp + c * akq
                K[p][p] = app - tt * apq
                K[q][q] = aqq + tt * apq
                K[p][q] = K[q][p] = zero
                for i in range(3):
                    vip = V[i][p]
                    viq = V[i][q]
                    V[i][p] = c * vip - s * viq
                    V[i][q] = s * vip + c * viq
        lam = [jnp.maximum(K[p][p], 0.0) for p in range(3)]
        inv = [jnp.sqrt(lam[p]) / (lam[p] + jnp.float32(1e-30))
               for p in range(3)]
        U = [[(S[i][0] * V[0][p] + S[i][1] * V[1][p] + S[i][2] * V[2][p])
              * inv[p] for p in range(3)] for i in range(3)]
        vm = [jnp.where(lam[0] <= lam[1],
                        jnp.where(lam[0] <= lam[2], V[i][0], V[i][2]),
                        jnp.where(lam[1] <= lam[2], V[i][1], V[i][2]))
              for i in range(3)]
        um = [jnp.where(lam[0] <= lam[1],
                        jnp.where(lam[0] <= lam[2], U[i][0], U[i][2]),
                        jnp.where(lam[1] <= lam[2], U[i][1], U[i][2]))
              for i in range(3)]
        R = [[(V[i][0] * U[j][0] + V[i][1] * U[j][1] + V[i][2] * U[j][2])
              - (one - d) * vm[i] * um[j] for j in range(3)]
             for i in range(3)]
        for p in range(3):
            tp = scm[p] - (R[p][0] * ycm[0] + R[p][1] * ycm[1]
                           + R[p][2] * ycm[2])
            for q in range(3):
                rr = R[p][q] - (one if p == q else zero)
                aw_ref[ch, 4 * p + q] = rr * vw
            aw_ref[ch, 4 * p + 3] = tp * vw
        aw_ref[ch, 12] = vw


def _loss_body(code_ref, f_ref, mom_ref, rows_ref, cols_ref, fv_ref,
               out_ref, aw_ref):
    b = pl.program_id(0)

    @pl.when(b == 0)
    def _():
        out_ref[0, 0] = 0.0
        _kabsch_prologue(mom_ref, rows_ref, cols_ref, fv_ref, aw_ref)

    ri = lax.broadcasted_iota(jnp.int32, (GRB, 1), 0)
    jcol = (ri & (W - 1)).astype(jnp.float32)
    icol = (lax.shift_right_logical(ri, 7)
            + b * (GRB // W)).astype(jnp.float32)
    krow = lax.broadcasted_iota(jnp.int32, (1, D), 1).astype(jnp.float32)
    code = code_ref[...].astype(jnp.int32)
    msk = jnp.zeros((GRB, D), jnp.float32)
    v = [jnp.zeros((GRB, D), jnp.float32) for _ in range(3)]
    for ch in range(NCH):
        yb = (lax.shift_right_logical(code, ch) & 1).astype(jnp.float32)
        msk = msk + yb * aw_ref[ch, 12]
        for p in range(3):
            rowpart = (aw_ref[ch, 4 * p + 0] * icol
                       + aw_ref[ch, 4 * p + 1] * jcol
                       + aw_ref[ch, 4 * p + 3])
            m = rowpart + aw_ref[ch, 4 * p + 2] * krow
            v[p] = v[p] + yb * m
    s2 = jnp.zeros((GRB, D), jnp.float32)
    for p in range(3):
        r = v[p] - msk * f_ref[p]
        s2 = s2 + r * r
    out_ref[0, 0] += jnp.sum(jnp.sqrt(s2))


def _run_loss(code, f3, mom, rows, cols, fv):
    return pl.pallas_call(
        _loss_body,
        grid=(NGRID,),
        in_specs=[
            pl.BlockSpec((GRB, D), lambda b: (b, 0)),
            pl.BlockSpec((3, GRB, D), lambda b: (0, b, 0)),
            pl.BlockSpec(memory_space=pltpu.SMEM),
            pl.BlockSpec((NCH * NS,), lambda b: (0,)),
            pl.BlockSpec((NCH * NS,), lambda b: (0,)),
            pl.BlockSpec((NCH * 3 * NS,), lambda b: (0,)),
        ],
        out_specs=pl.BlockSpec(memory_space=pltpu.SMEM),
        out_shape=jax.ShapeDtypeStruct((1, 1), jnp.float32),
        scratch_shapes=[pltpu.SMEM((NCH, 16), jnp.float32)],
        compiler_params=pltpu.CompilerParams(
            dimension_semantics=("arbitrary",)),
    )(code, f3, mom, rows, cols, fv)


def _sample_targets(valid, cnt_y):
    count = cnt_y.astype(jnp.int32)
    rank = jnp.cumsum(valid.astype(jnp.int32)) - 1
    key = jax.random.key(42)
    keys = jax.vmap(lambda r: jax.random.fold_in(key, r))(rank)
    idx = jax.vmap(
        lambda k, m: jax.random.randint(k, (NS,), 0, m)
    )(keys, jnp.maximum(count, 1))
    return (idx + 1).astype(jnp.int32)


def kernel(y_source_oh, source_oh, flow, neg_flow):
    y2d = y_source_oh.reshape(4 * NROW, D)
    s2d = source_oh.reshape(4 * NROW, D)
    f3 = flow.reshape(3, NROW, D)
    f2d = flow.reshape(3 * NROW, D)

    rowcnt, mom, blk, code = _run_stats(y2d, s2d)

    valid = (mom[:, 0] > 100.0) & (mom[:, 4] > 100.0)
    targets = _sample_targets(valid, mom[:, 0])

    rows, cols, fv = _run_sample(rowcnt.reshape(NCH * NROW),
                                 blk.reshape(NCH * NBLK),
                                 targets.reshape(NCH * NS), y2d, f2d)
    total = _run_loss(code, f3, mom, rows, cols, fv)
    return (total[0, 0] / NVOX).astype(jnp.float32)


# speedup vs baseline: 9.7289x; 1.0031x over previous
import functools

import jax
import jax.numpy as jnp
from jax import lax
from jax.experimental import pallas as pl
from jax.experimental.pallas import tpu as pltpu
from jax.experimental.pallas import tpu_sc as plsc

H, W, D = 64, 128, 128
NROW = H * W
NBLK = 64
RPB = NROW // NBLK
NGRID = 8
GRB = NROW // NGRID
NCH = 3
NS = 16
NVOX = H * W * D


def _stats_body(y1_ref, y2_ref, y3_ref, s1_ref, s2_ref, s3_ref,
                rowcnt_ref, mom_ref, blk_ref, code_ref):
    b = pl.program_id(0)

    @pl.when(b == 0)
    def _():
        for ch in range(NCH):
            for q in range(8):
                mom_ref[ch, q] = 0.0

    ri = lax.broadcasted_iota(jnp.int32, (GRB,), 0)
    jr = (ri & (W - 1)).astype(jnp.float32)
    ir = (lax.shift_right_logical(ri, 7) + b * (GRB // W)).astype(jnp.float32)
    kv = lax.broadcasted_iota(jnp.int32, (D,), 0).astype(jnp.float32)
    ys = [y1_ref[...], y2_ref[...], y3_ref[...]]
    ss = [s1_ref[...], s2_ref[...], s3_ref[...]]
    code = (ys[0] + 2.0 * ys[1] + 4.0 * ys[2]).astype(jnp.int32)
    code_ref[...] = code.astype(jnp.int8)
    for ch in range(NCH):
        yb = ys[ch]
        sb = ss[ch]
        rs = jnp.sum(yb, axis=1)
        rss = jnp.sum(sb, axis=1)
        rowcnt_ref[ch, :] = rs.astype(jnp.int32)
        cy = jnp.float32(0.0)
        for q in range(GRB // RPB):
            cyp = jnp.sum(rs[q * RPB:(q + 1) * RPB])
            blk_ref[ch, (GRB // RPB) * b + q] = cyp.astype(jnp.int32)
            cy = cy + cyp
        mom_ref[ch, 0] += cy
        mom_ref[ch, 1] += jnp.sum(ir * rs)
        mom_ref[ch, 2] += jnp.sum(jr * rs)
        mom_ref[ch, 3] += jnp.sum(jnp.sum(yb, axis=0) * kv)
        mom_ref[ch, 4] += jnp.sum(rss)
        mom_ref[ch, 5] += jnp.sum(ir * rss)
        mom_ref[ch, 6] += jnp.sum(jr * rss)
        mom_ref[ch, 7] += jnp.sum(jnp.sum(sb, axis=0) * kv)


def _chan_spec(ch):
    return pl.BlockSpec((GRB, D), lambda b, c=ch: ((c + 1) * (NROW // GRB) + b, 0))


def _run_stats(y2d, s2d):
    return pl.pallas_call(
        _stats_body,
        grid=(NGRID,),
        in_specs=[_chan_spec(0), _chan_spec(1), _chan_spec(2),
                  _chan_spec(0), _chan_spec(1), _chan_spec(2)],
        out_specs=[
            pl.BlockSpec((NCH, GRB), lambda b: (0, b)),
            pl.BlockSpec(memory_space=pltpu.SMEM),
            pl.BlockSpec(memory_space=pltpu.SMEM),
            pl.BlockSpec((GRB, D), lambda b: (b, 0)),
        ],
        out_shape=[
            jax.ShapeDtypeStruct((NCH, NROW), jnp.int32),
            jax.ShapeDtypeStruct((NCH, 8), jnp.float32),
            jax.ShapeDtypeStruct((NCH, NBLK), jnp.int32),
            jax.ShapeDtypeStruct((NROW, D), jnp.int8),
        ],
        compiler_params=pltpu.CompilerParams(
            dimension_semantics=("arbitrary",)),
    )(y2d, y2d, y2d, s2d, s2d, s2d)


def _sample_body(rowcnt_hbm, blkcnt_hbm, tgt_hbm, y_hbm, f_hbm,
                 rows_out, cols_out, fv_out,
                 rc_v, bc_v, tg_v, idx_v, yrows_v, frow_v, st_v, sti_v, sem):
    cid = lax.axis_index("c")
    sid = lax.axis_index("s")
    wid = sid * 2 + cid

    @pl.when(wid < NCH)
    def _():
        ch = wid
        pltpu.sync_copy(rowcnt_hbm.at[pl.ds(ch * NROW, NROW)], rc_v)
        pltpu.sync_copy(blkcnt_hbm.at[pl.ds(ch * NBLK, NBLK)], bc_v)
        pltpu.sync_copy(tgt_hbm.at[pl.ds(ch * NS, NS)], tg_v)
        lanes = lax.iota(jnp.int32, 16)
        t = tg_v[...]

        z16 = jnp.zeros((16,), jnp.int32)

        def blk_body(b, carry):
            cum, blk, cb = carry
            v = plsc.load_gather(bc_v, [jnp.broadcast_to(b, (16,))])
            newcum = cum + v
            hit = (newcum >= t) & (cum < t)
            blk = jnp.where(hit, b, blk)
            cb = jnp.where(hit, cum, cb)
            return newcum, blk, cb

        _, blk, cb = lax.fori_loop(0, NBLK, blk_body, (z16, z16, z16))

        rowbase = blk * RPB

        def row_body(r, carry):
            cum, row, rstart = carry
            v = plsc.load_gather(rc_v, [rowbase + r])
            newcum = cum + v
            hit = (newcum >= t) & (cum < t)
            row = jnp.where(hit, rowbase + r, row)
            rstart = jnp.where(hit, cum, rstart)
            return newcum, row, rstart

        _, row, rstart = lax.fori_loop(0, RPB, row_body, (cb, z16, z16))
        t_local = t - rstart

        idx_v[...] = row + (ch + 1) * NROW
        pltpu.async_copy(y_hbm.at[idx_v], yrows_v, sem).wait()

        def col_body(c, carry):
            cum2, col = carry
            cs = jnp.broadcast_to(c, (16,))
            vals = plsc.load_gather(yrows_v, [lanes, cs])
            isnz = vals > 0.5
            cnew = cum2 + isnz.astype(jnp.int32)
            col = jnp.where(isnz & (cnew == t_local), cs, col)
            return cnew, col

        _, col = lax.fori_loop(0, D, col_body, (z16, z16))

        sti_v[...] = row
        pltpu.sync_copy(sti_v, rows_out.at[pl.ds(ch * NS, NS)])
        sti_v[...] = col
        pltpu.sync_copy(sti_v, cols_out.at[pl.ds(ch * NS, NS)])

        for comp in range(3):
            idx_v[...] = row + comp * NROW
            pltpu.async_copy(f_hbm.at[idx_v], frow_v, sem).wait()
            st_v[...] = plsc.load_gather(frow_v, [lanes, col])
            pltpu.sync_copy(st_v, fv_out.at[pl.ds((ch * 3 + comp) * NS, NS)])


def _run_sample(rowcnt_i, blkcnt_i, targets, y2d, f2d):
    mesh = plsc.VectorSubcoreMesh(core_axis_name="c", subcore_axis_name="s")
    fn = pl.kernel(
        _sample_body,
        out_type=[
            jax.ShapeDtypeStruct((NCH * NS,), jnp.int32),
            jax.ShapeDtypeStruct((NCH * NS,), jnp.int32),
            jax.ShapeDtypeStruct((NCH * 3 * NS,), jnp.float32),
        ],
        mesh=mesh,
        compiler_params=pltpu.CompilerParams(needs_layout_passes=False),
        scratch_types=[
            pltpu.VMEM((NROW,), jnp.int32),
            pltpu.VMEM((NBLK,), jnp.int32),
            pltpu.VMEM((NS,), jnp.int32),
            pltpu.VMEM((NS,), jnp.int32),
            pltpu.VMEM((NS, D), jnp.float32),
            pltpu.VMEM((NS, D), jnp.float32),
            pltpu.VMEM((NS,), jnp.float32),
            pltpu.VMEM((NS,), jnp.int32),
            pltpu.SemaphoreType.DMA,
        ],
    )
    return fn(rowcnt_i, blkcnt_i, targets, y2d, f2d)


def _kabsch_prologue(mom_ref, rows_ref, cols_ref, fv_ref, aw_ref):
    one = jnp.float32(1.0)
    zero = jnp.float32(0.0)
    for ch in range(NCH):
        cnt_y = mom_ref[ch, 0]
        cnt_s = mom_ref[ch, 4]
        vw = jnp.where((cnt_y > 100.0) & (cnt_s > 100.0), one, zero)
        ty = jnp.where(cnt_y > 0.0, cnt_y, one)
        ts = jnp.where(cnt_s > 0.0, cnt_s, one)
        ycm = [mom_ref[ch, 1 + p] / ty for p in range(3)]
        scm = [mom_ref[ch, 5 + p] / ts for p in range(3)]
        rowv = rows_ref[pl.ds(ch * NS, NS)]
        colv = cols_ref[pl.ds(ch * NS, NS)]
        src = [lax.shift_right_logical(rowv, 7).astype(jnp.float32),
               (rowv & (W - 1)).astype(jnp.float32),
               colv.astype(jnp.float32)]
        X = [src[p] - ycm[p] for p in range(3)]
        Y = [src[p] + fv_ref[pl.ds((ch * 3 + p) * NS, NS)] - scm[p]
             for p in range(3)]
        S = [[jnp.sum(X[p] * Y[q]) for q in range(3)] for p in range(3)]
        detS = (S[0][0] * (S[1][1] * S[2][2] - S[1][2] * S[2][1])
                - S[0][1] * (S[1][0] * S[2][2] - S[1][2] * S[2][0])
                + S[0][2] * (S[1][0] * S[2][1] - S[1][1] * S[2][0]))
        d = jnp.where(detS >= 0.0, one, -one)
        K = [[S[0][a] * S[0][b] + S[1][a] * S[1][b] + S[2][a] * S[2][b]
              for b in range(3)] for a in range(3)]
        V = [[one if i == j else zero for j in range(3)] for i in range(3)]
        for _ in range(6):
            for (p, q) in ((0, 1), (0, 2), (1, 2)):
                apq = K[p][q]
                app = K[p][p]
                aqq = K[q][q]
                apq_s = jnp.where(jnp.abs(apq) > 1e-30, apq,
                                  jnp.float32(1e-30))
                tau = (aqq - app) / (2.0 * apq_s)
                tt = jnp.sign(tau) / (jnp.abs(tau)
                                      + jnp.sqrt(1.0 + tau * tau))
                c = 1.0 / jnp.sqrt(1.0 + tt * tt)
                s = tt * c
                r = 3 - p - q
                akp = K[r][p]
                akq = K[r][q]
                K[r][p] = K[p][r] = c * akp - s * akq
                K[r][q] = K[q][r] = s * akp + c * akq
                K[p][p] = app - tt * apq
                K[q][q] = aqq + tt * apq
                K[p][q] = K[q][p] = zero
                for i in range(3):
                    vip = V[i][p]
                    viq = V[i][q]
                    V[i][p] = c * vip - s * viq
                    V[i][q] = s * vip + c * viq
        lam = [jnp.maximum(K[p][p], 0.0) for p in range(3)]
        inv = [jnp.sqrt(lam[p]) / (lam[p] + jnp.float32(1e-30))
               for p in range(3)]
        U = [[(S[i][0] * V[0][p] + S[i][1] * V[1][p] + S[i][2] * V[2][p])
              * inv[p] for p in range(3)] for i in range(3)]
        vm = [jnp.where(lam[0] <= lam[1],
                        jnp.where(lam[0] <= lam[2], V[i][0], V[i][2]),
                        jnp.where(lam[1] <= lam[2], V[i][1], V[i][2]))
              for i in range(3)]
        um = [jnp.where(lam[0] <= lam[1],
                        jnp.where(lam[0] <= lam[2], U[i][0], U[i][2]),
                        jnp.where(lam[1] <= lam[2], U[i][1], U[i][2]))
              for i in range(3)]
        R = [[(V[i][0] * U[j][0] + V[i][1] * U[j][1] + V[i][2] * U[j][2])
              - (one - d) * vm[i] * um[j] for j in range(3)]
             for i in range(3)]
        for p in range(3):
            tp = scm[p] - (R[p][0] * ycm[0] + R[p][1] * ycm[1]
                           + R[p][2] * ycm[2])
            for q in range(3):
                rr = R[p][q] - (one if p == q else zero)
                aw_ref[ch, 4 * p + q] = rr * vw
            aw_ref[ch, 4 * p + 3] = tp * vw
        aw_ref[ch, 12] = vw


def _loss_body(code_ref, f_ref, mom_ref, rows_ref, cols_ref, fv_ref,
               out_ref, aw_ref):
    b = pl.program_id(0)

    @pl.when(b == 0)
    def _():
        out_ref[0, 0] = 0.0
        _kabsch_prologue(mom_ref, rows_ref, cols_ref, fv_ref, aw_ref)

    ri = lax.broadcasted_iota(jnp.int32, (GRB, 1), 0)
    jcol = (ri & (W - 1)).astype(jnp.float32)
    icol = (lax.shift_right_logical(ri, 7)
            + b * (GRB // W)).astype(jnp.float32)
    krow = lax.broadcasted_iota(jnp.int32, (1, D), 1).astype(jnp.float32)
    code = code_ref[...].astype(jnp.int32)
    msk = jnp.zeros((GRB, D), jnp.float32)
    v = [jnp.zeros((GRB, D), jnp.float32) for _ in range(3)]
    for ch in range(NCH):
        yb = (lax.shift_right_logical(code, ch) & 1).astype(jnp.float32)
        msk = msk + yb * aw_ref[ch, 12]
        for p in range(3):
            rowpart = (aw_ref[ch, 4 * p + 0] * icol
                       + aw_ref[ch, 4 * p + 1] * jcol
                       + aw_ref[ch, 4 * p + 3])
            m = rowpart + aw_ref[ch, 4 * p + 2] * krow
            v[p] = v[p] + yb * m
    s2 = jnp.zeros((GRB, D), jnp.float32)
    for p in range(3):
        r = v[p] - msk * f_ref[p]
        s2 = s2 + r * r
    out_ref[0, 0] += jnp.sum(jnp.sqrt(s2))


def _run_loss(code, f3, mom, rows, cols, fv):
    return pl.pallas_call(
        _loss_body,
        grid=(NGRID,),
        in_specs=[
            pl.BlockSpec((GRB, D), lambda b: (b, 0)),
            pl.BlockSpec((3, GRB, D), lambda b: (0, b, 0)),
            pl.BlockSpec(memory_space=pltpu.SMEM),
            pl.BlockSpec((NCH * NS,), lambda b: (0,)),
            pl.BlockSpec((NCH * NS,), lambda b: (0,)),
            pl.BlockSpec((NCH * 3 * NS,), lambda b: (0,)),
        ],
        out_specs=pl.BlockSpec(memory_space=pltpu.SMEM),
        out_shape=jax.ShapeDtypeStruct((1, 1), jnp.float32),
        scratch_shapes=[pltpu.SMEM((NCH, 16), jnp.float32)],
        compiler_params=pltpu.CompilerParams(
            dimension_semantics=("arbitrary",)),
    )(code, f3, mom, rows, cols, fv)


def _sample_targets(valid, cnt_y):
    count = cnt_y.astype(jnp.int32)
    rank = jnp.cumsum(valid.astype(jnp.int32)) - 1
    key = jax.random.key(42)
    keys = jax.vmap(lambda r: jax.random.fold_in(key, r))(rank)
    idx = jax.vmap(
        lambda k, m: jax.random.randint(k, (NS,), 0, m)
    )(keys, jnp.maximum(count, 1))
    return (idx + 1).astype(jnp.int32)


def kernel(y_source_oh, source_oh, flow, neg_flow):
    y2d = y_source_oh.reshape(4 * NROW, D)
    s2d = source_oh.reshape(4 * NROW, D)
    f3 = flow.reshape(3, NROW, D)
    f2d = flow.reshape(3 * NROW, D)

    rowcnt, mom, blk, code = _run_stats(y2d, s2d)

    valid = (mom[:, 0] > 100.0) & (mom[:, 4] > 100.0)
    targets = _sample_targets(valid, mom[:, 0])

    rows, cols, fv = _run_sample(rowcnt.reshape(NCH * NROW),
                                 blk.reshape(NCH * NBLK),
                                 targets.reshape(NCH * NS), y2d, f2d)
    total = _run_loss(code, f3, mom, rows, cols, fv)
    return (total[0, 0] / NVOX).astype(jnp.float32)
